# Initial kernel scaffold; baseline (speedup 1.0000x reference)
import jax
import jax.numpy as jnp
from jax.experimental import pallas as pl


def kernel(x, edge_index, W1, b1, W2, b2):
    raise NotImplementedError("write your pallas kernel here")


# speedup vs baseline: 47.9235x; 47.9235x over previous
import functools

import jax
import jax.numpy as jnp
from jax import lax
from jax.experimental import pallas as pl
from jax.experimental.pallas import tpu as pltpu
from jax.experimental.pallas import tpu_sc as plsc

NC = 2
NS = 16
NW = NC * NS
LANE = 128
K = 16


def _sc_hist(col2d, z_tile, npad, wn):
    gpw = wn * K
    rpt = npad // NS

    @functools.partial(
        pl.kernel,
        out_type=jax.ShapeDtypeStruct((NC * npad,), jnp.float32),
        mesh=plsc.VectorSubcoreMesh(core_axis_name="c", subcore_axis_name="s"),
        scratch_types=[
            pltpu.VMEM_SHARED((npad,), jnp.float32),
            pltpu.VMEM((K, LANE), jnp.int32),
            pltpu.VMEM((LANE,), jnp.float32),
            pltpu.VMEM((rpt,), jnp.float32),
            pltpu.SemaphoreType.DMA,
        ],
    )
    def hist(col_hbm, z_hbm, out_hbm, acc, col_buf, ones, zbuf, sem):
        cid = lax.axis_index("c")
        sid = lax.axis_index("s")
        w = cid * NS + sid
        pltpu.sync_copy(z_hbm, zbuf)
        pltpu.sync_copy(zbuf, acc.at[pl.ds(sid * rpt, rpt)])
        for t in range(LANE // 16):
            ones[pl.ds(t * 16, 16)] = jnp.ones((16,), jnp.float32)
        plsc.subcore_barrier()

        def body(win, carry):
            g0 = w * gpw + win * K
            pltpu.sync_copy(col_hbm.at[pl.ds(g0, K)], col_buf)
            descs = [
                pltpu.async_copy(ones, acc.at[col_buf.at[j]], sem, add=True)
                for j in range(K)
            ]
            for d in descs:
                d.wait()
            return carry

        lax.fori_loop(0, wn, body, 0)
        plsc.subcore_barrier()
        pltpu.sync_copy(acc.at[pl.ds(sid * rpt, rpt)], zbuf)
        pltpu.sync_copy(zbuf, out_hbm.at[pl.ds(cid * npad + sid * rpt, rpt)])

    return hist(col2d, z_tile)


def _sc_prop(row2d, col2d, y, z_tile, npad, wn, F):
    gpw = wn * K
    rpt = npad // NS

    @functools.partial(
        pl.kernel,
        out_type=jax.ShapeDtypeStruct((NC * npad, F), jnp.float32),
        mesh=plsc.VectorSubcoreMesh(core_axis_name="c", subcore_axis_name="s"),
        compiler_params=pltpu.CompilerParams(use_tc_tiling_on_sc=False),
        scratch_types=[
            pltpu.VMEM_SHARED((npad, F), jnp.float32),
            pltpu.VMEM((K, LANE), jnp.int32),
            pltpu.VMEM((K, LANE), jnp.int32),
            pltpu.VMEM((K * LANE, F), jnp.float32),
            pltpu.VMEM((rpt, F), jnp.float32),
            pltpu.SemaphoreType.DMA,
            pltpu.SemaphoreType.DMA,
        ],
    )
    def prop(row_hbm, col_hbm, y_hbm, z_hbm, out_hbm,
             acc, row_buf, col_buf, gbuf, zbuf, gsem, ssem):
        cid = lax.axis_index("c")
        sid = lax.axis_index("s")
        w = cid * NS + sid
        pltpu.sync_copy(z_hbm, zbuf)
        pltpu.sync_copy(zbuf, acc.at[pl.ds(sid * rpt, rpt)])
        plsc.subcore_barrier()

        def body(win, carry):
            g0 = w * gpw + win * K
            pltpu.sync_copy(row_hbm.at[pl.ds(g0, K)], row_buf)
            pltpu.sync_copy(col_hbm.at[pl.ds(g0, K)], col_buf)
            gd = [
                pltpu.async_copy(y_hbm.at[row_buf.at[j]],
                                 gbuf.at[pl.ds(j * LANE, LANE)], gsem)
                for j in range(K)
            ]
            for d in gd:
                d.wait()
            sd = [
                pltpu.async_copy(gbuf.at[pl.ds(j * LANE, LANE)],
                                 acc.at[col_buf.at[j]], ssem, add=True)
                for j in range(K)
            ]
            for d in sd:
                d.wait()
            return carry

        lax.fori_loop(0, wn, body, 0)
        plsc.subcore_barrier()
        pltpu.sync_copy(acc.at[pl.ds(sid * rpt, rpt)], zbuf)
        pltpu.sync_copy(zbuf, out_hbm.at[pl.ds(cid * npad + sid * rpt, rpt)])

    return prop(row2d, col2d, y, z_tile)


_TCB = 2048


def _tc_pre(x, W1, degp, N, npad):
    Fin, Fout = W1.shape
    degp2 = degp.reshape(NC, npad)

    def body(x_ref, w_ref, dp_ref, dis_ref, y1_ref):
        deg = dp_ref[0, :] + dp_ref[1, :] + 1.0
        dis = lax.rsqrt(deg)[:, None]
        xw = jnp.dot(x_ref[...], w_ref[...], preferred_element_type=jnp.float32)
        dis_ref[...] = dis
        y1_ref[...] = xw * dis

    return pl.pallas_call(
        body,
        grid=(pl.cdiv(N, _TCB),),
        in_specs=[
            pl.BlockSpec((_TCB, Fin), lambda i: (i, 0)),
            pl.BlockSpec((Fin, Fout), lambda i: (0, 0)),
            pl.BlockSpec((NC, _TCB), lambda i: (0, i)),
        ],
        out_specs=[
            pl.BlockSpec((_TCB, 1), lambda i: (i, 0)),
            pl.BlockSpec((_TCB, Fout), lambda i: (i, 0)),
        ],
        out_shape=[
            jax.ShapeDtypeStruct((N, 1), jnp.float32),
            jax.ShapeDtypeStruct((N, Fout), jnp.float32),
        ],
    )(x, W1, degp2)


def _tc_mid(acc1, y1, dis, W2, b1, N, npad):
    Fin, Fout = W2.shape
    acc = acc1.reshape(NC, npad, Fin)

    def body(a_ref, y1_ref, dis_ref, w_ref, b_ref, y2_ref):
        p = a_ref[0] + a_ref[1]
        dis = dis_ref[...]
        h = jnp.maximum(dis * (p + y1_ref[...]) + b_ref[...], 0.0)
        y2_ref[...] = jnp.dot(h, w_ref[...],
                              preferred_element_type=jnp.float32) * dis

    return pl.pallas_call(
        body,
        grid=(pl.cdiv(N, _TCB),),
        in_specs=[
            pl.BlockSpec((NC, _TCB, Fin), lambda i: (0, i, 0)),
            pl.BlockSpec((_TCB, Fin), lambda i: (i, 0)),
            pl.BlockSpec((_TCB, 1), lambda i: (i, 0)),
            pl.BlockSpec((Fin, Fout), lambda i: (0, 0)),
            pl.BlockSpec((1, Fin), lambda i: (0, 0)),
        ],
        out_specs=pl.BlockSpec((_TCB, Fout), lambda i: (i, 0)),
        out_shape=jax.ShapeDtypeStruct((N, Fout), jnp.float32),
    )(acc, y1, dis, W2, b1.reshape(1, Fin))


def _tc_post(acc2, y2, dis, b2, N, npad, Fout):
    F = y2.shape[1]
    acc = acc2.reshape(NC, npad, F)

    def body(a_ref, y2_ref, dis_ref, b_ref, o_ref):
        full = dis_ref[...] * (a_ref[0] + a_ref[1] + y2_ref[...])
        o_ref[...] = full[:, :Fout] + b_ref[...]

    return pl.pallas_call(
        body,
        grid=(pl.cdiv(N, _TCB),),
        in_specs=[
            pl.BlockSpec((NC, _TCB, F), lambda i: (0, i, 0)),
            pl.BlockSpec((_TCB, F), lambda i: (i, 0)),
            pl.BlockSpec((_TCB, 1), lambda i: (i, 0)),
            pl.BlockSpec((1, Fout), lambda i: (0, 0)),
        ],
        out_specs=pl.BlockSpec((_TCB, Fout), lambda i: (i, 0)),
        out_shape=jax.ShapeDtypeStruct((N, Fout), jnp.float32),
    )(acc, y2, dis, b2.reshape(1, Fout))


def kernel(x, edge_index, W1, b1, W2, b2):
    N = x.shape[0]
    E = edge_index.shape[1]
    npad = ((N + 64 + 127) // 128) * 128

    G = -(-E // LANE)
    wn = -(-G // (NW * K))
    Gp = NW * K * wn
    pad = Gp * LANE - E

    row = edge_index[0].astype(jnp.int32)
    col = edge_index[1].astype(jnp.int32)
    ar = jnp.arange(pad, dtype=jnp.int32)
    row2d = jnp.concatenate([row, ar % 64]).reshape(Gp, LANE)
    col2d = jnp.concatenate([col, N + (ar % 64)]).reshape(Gp, LANE)

    rpt = npad // NS
    z1 = jnp.zeros((rpt,), jnp.float32)
    z8 = jnp.zeros((rpt, W1.shape[1]), jnp.float32)

    F2 = W2.shape[1]
    W2p = jnp.pad(W2, ((0, 0), (0, 8 - F2)))

    degp = _sc_hist(col2d, z1, npad, wn)
    dis, y1 = _tc_pre(x, W1, degp, N, npad)
    acc1 = _sc_prop(row2d, col2d, y1, z8, npad, wn, W1.shape[1])
    y2 = _tc_mid(acc1, y1, dis, W2p, b1, N, npad)
    acc2 = _sc_prop(row2d, col2d, y2, z8, npad, wn, 8)
    return _tc_post(acc2, y2, dis, b2, N, npad, F2)


# speedup vs baseline: 61.5655x; 1.2847x over previous
import functools

import jax
import jax.numpy as jnp
from jax import lax
from jax.experimental import pallas as pl
from jax.experimental.pallas import tpu as pltpu
from jax.experimental.pallas import tpu_sc as plsc

NC = 2
NS = 16
NW = NC * NS
LANE = 128
K = 16
PK = 16


def _sc_hist(col2d, z_tile, npad, wn):
    gpw = wn * K
    rpt = npad // NS

    @functools.partial(
        pl.kernel,
        out_type=jax.ShapeDtypeStruct((NC * npad,), jnp.float32),
        mesh=plsc.VectorSubcoreMesh(core_axis_name="c", subcore_axis_name="s"),
        scratch_types=[
            pltpu.VMEM_SHARED((npad,), jnp.float32),
            pltpu.VMEM((K, LANE), jnp.int32),
            pltpu.VMEM((LANE,), jnp.float32),
            pltpu.VMEM((rpt,), jnp.float32),
            pltpu.SemaphoreType.DMA,
        ],
        compiler_params=pltpu.CompilerParams(use_tc_tiling_on_sc=False),
    )
    def hist(col_hbm, z_hbm, out_hbm, acc, col_buf, ones, zbuf, sem):
        cid = lax.axis_index("c")
        sid = lax.axis_index("s")
        w = cid * NS + sid
        pltpu.sync_copy(z_hbm, zbuf)
        pltpu.sync_copy(zbuf, acc.at[pl.ds(sid * rpt, rpt)])
        for t in range(LANE // 16):
            ones[pl.ds(t * 16, 16)] = jnp.ones((16,), jnp.float32)
        plsc.subcore_barrier()

        def body(win, carry):
            g0 = w * gpw + win * K
            pltpu.sync_copy(col_hbm.at[pl.ds(g0, K)], col_buf)
            descs = [
                pltpu.async_copy(ones, acc.at[col_buf.at[j]], sem, add=True)
                for j in range(K)
            ]
            for d in descs:
                d.wait()
            return carry

        lax.fori_loop(0, wn, body, 0)
        plsc.subcore_barrier()
        pltpu.sync_copy(acc.at[pl.ds(sid * rpt, rpt)], zbuf)
        pltpu.sync_copy(zbuf, out_hbm.at[pl.ds(cid * npad + sid * rpt, rpt)])

    return hist(col2d, z_tile)


def _sc_prop(row2d, col2d, y, z_tile, npad, wn, F):
    gpw = wn * K
    rpt = npad // NS

    @functools.partial(
        pl.kernel,
        out_type=jax.ShapeDtypeStruct((NC * npad, F), jnp.float32),
        mesh=plsc.VectorSubcoreMesh(core_axis_name="c", subcore_axis_name="s"),
        scratch_types=[
            pltpu.VMEM_SHARED((npad, F), jnp.float32),
            pltpu.VMEM((K, LANE), jnp.int32),
            pltpu.VMEM((K, LANE), jnp.int32),
            pltpu.VMEM((K * LANE, F), jnp.float32),
            pltpu.VMEM((rpt, F), jnp.float32),
            pltpu.SemaphoreType.DMA,
            pltpu.SemaphoreType.DMA,
        ],
        compiler_params=pltpu.CompilerParams(use_tc_tiling_on_sc=False),
    )
    def prop(row_hbm, col_hbm, y_hbm, z_hbm, out_hbm,
             acc, row_buf, col_buf, gbuf, zbuf, gsem, ssem):
        cid = lax.axis_index("c")
        sid = lax.axis_index("s")
        w = cid * NS + sid
        pltpu.sync_copy(z_hbm, zbuf)
        pltpu.sync_copy(zbuf, acc.at[pl.ds(sid * rpt, rpt)])
        plsc.subcore_barrier()

        def body(win, carry):
            g0 = w * gpw + win * K
            pltpu.sync_copy(row_hbm.at[pl.ds(g0, K)], row_buf)
            pltpu.sync_copy(col_hbm.at[pl.ds(g0, K)], col_buf)
            gd = [
                pltpu.async_copy(y_hbm.at[row_buf.at[j]],
                                 gbuf.at[pl.ds(j * LANE, LANE)], gsem)
                for j in range(K)
            ]
            sd = []
            for j in range(K):
                gd[j].wait()
                sd.append(pltpu.async_copy(gbuf.at[pl.ds(j * LANE, LANE)],
                                           acc.at[col_buf.at[j]], ssem,
                                           add=True))
            for d in sd:
                d.wait()
            return carry

        lax.fori_loop(0, wn, body, 0)
        plsc.subcore_barrier()
        pltpu.sync_copy(acc.at[pl.ds(sid * rpt, rpt)], zbuf)
        pltpu.sync_copy(zbuf, out_hbm.at[pl.ds(cid * npad + sid * rpt, rpt)])

    return prop(row2d, col2d, y, z_tile)


_TCB = 800


def _tc_pre(xP, W1B, degpP, npadP):
    def body(x_ref, w_ref, dp_ref, dis_ref, y1_ref):
        deg = dp_ref[0] + dp_ref[1] + 1.0
        dis = lax.rsqrt(deg)
        xw = jnp.dot(x_ref[...], w_ref[...], preferred_element_type=jnp.float32)
        dis_ref[...] = dis
        y1_ref[...] = xw * dis

    return pl.pallas_call(
        body,
        grid=(npadP // _TCB,),
        in_specs=[
            pl.BlockSpec((_TCB, 128), lambda i: (i, 0)),
            pl.BlockSpec((128, 128), lambda i: (0, 0)),
            pl.BlockSpec((NC, _TCB, 128), lambda i: (0, i, 0)),
        ],
        out_specs=[
            pl.BlockSpec((_TCB, 128), lambda i: (i, 0)),
            pl.BlockSpec((_TCB, 128), lambda i: (i, 0)),
        ],
        out_shape=[
            jax.ShapeDtypeStruct((npadP, 128), jnp.float32),
            jax.ShapeDtypeStruct((npadP, 128), jnp.float32),
        ],
    )(xP, W1B, degpP)


def _tc_mid(acc1P, y1P, disP, W2B, b1P, npadP):
    def body(a_ref, y1_ref, dis_ref, w_ref, b_ref, y2_ref):
        dis = dis_ref[...]
        h = jnp.maximum(dis * (a_ref[0] + a_ref[1] + y1_ref[...]) + b_ref[...],
                        0.0)
        y2_ref[...] = jnp.dot(h, w_ref[...],
                              preferred_element_type=jnp.float32) * dis

    return pl.pallas_call(
        body,
        grid=(npadP // _TCB,),
        in_specs=[
            pl.BlockSpec((NC, _TCB, 128), lambda i: (0, i, 0)),
            pl.BlockSpec((_TCB, 128), lambda i: (i, 0)),
            pl.BlockSpec((_TCB, 128), lambda i: (i, 0)),
            pl.BlockSpec((128, 128), lambda i: (0, 0)),
            pl.BlockSpec((1, 128), lambda i: (0, 0)),
        ],
        out_specs=pl.BlockSpec((_TCB, 128), lambda i: (i, 0)),
        out_shape=jax.ShapeDtypeStruct((npadP, 128), jnp.float32),
    )(acc1P, y1P, disP, W2B, b1P)


def _tc_post(acc2P, y2P, disP, b2P, npadP):
    def body(a_ref, y2_ref, dis_ref, b_ref, o_ref):
        o_ref[...] = (dis_ref[...] * (a_ref[0] + a_ref[1] + y2_ref[...])
                      + b_ref[...])

    return pl.pallas_call(
        body,
        grid=(npadP // _TCB,),
        in_specs=[
            pl.BlockSpec((NC, _TCB, 128), lambda i: (0, i, 0)),
            pl.BlockSpec((_TCB, 128), lambda i: (i, 0)),
            pl.BlockSpec((_TCB, 128), lambda i: (i, 0)),
            pl.BlockSpec((1, 128), lambda i: (0, 0)),
        ],
        out_specs=pl.BlockSpec((_TCB, 128), lambda i: (i, 0)),
        out_shape=jax.ShapeDtypeStruct((npadP, 128), jnp.float32),
    )(acc2P, y2P, disP, b2P)


def kernel(x, edge_index, W1, b1, W2, b2):
    N = x.shape[0]
    E = edge_index.shape[1]
    npad = 102400 if N == 100000 else ((N + 64 + 16 * _TCB - 1)
                                       // (16 * _TCB)) * (16 * _TCB)
    npadP = npad // PK

    G = -(-E // LANE)
    wn = -(-G // (NW * K))
    Gp = NW * K * wn
    pad = Gp * LANE - E

    row = edge_index[0].astype(jnp.int32)
    col = edge_index[1].astype(jnp.int32)
    ar = jnp.arange(pad, dtype=jnp.int32)
    row2d = jnp.concatenate([row, ar % 64]).reshape(Gp, LANE)
    col2d = jnp.concatenate([col, N + (ar % 64)]).reshape(Gp, LANE)

    rpt = npad // NS
    z1 = jnp.zeros((rpt,), jnp.float32)
    z8 = jnp.zeros((rpt, 8), jnp.float32)

    xP = jnp.pad(x, ((0, npad - N), (0, 8 - x.shape[1]))).reshape(npadP, 128)
    W1B = jnp.kron(jnp.eye(PK, dtype=jnp.float32),
                   jnp.pad(W1, ((0, 8 - W1.shape[0]), (0, 8 - W1.shape[1]))))
    F2 = W2.shape[1]
    W2B = jnp.kron(jnp.eye(PK, dtype=jnp.float32),
                   jnp.pad(W2, ((0, 8 - W2.shape[0]), (0, 8 - F2))))
    b1P = jnp.tile(jnp.pad(b1, (0, 8 - b1.shape[0])), PK).reshape(1, 128)
    b2P = jnp.tile(jnp.pad(b2, (0, 8 - F2)), PK).reshape(1, 128)

    degp = _sc_hist(col2d, z1, npad, wn)
    degpP = jnp.broadcast_to(degp.reshape(NC, npad, 1),
                             (NC, npad, 8)).reshape(NC, npadP, 128)
    disP, y1P = _tc_pre(xP, W1B, degpP, npadP)
    acc1 = _sc_prop(row2d, col2d, y1P.reshape(npad, 8), z8, npad, wn, 8)
    y2P = _tc_mid(acc1.reshape(NC, npadP, 128), y1P, disP, W2B, b1P, npadP)
    acc2 = _sc_prop(row2d, col2d, y2P.reshape(npad, 8), z8, npad, wn, 8)
    outP = _tc_post(acc2.reshape(NC, npadP, 128), y2P, disP, b2P, npadP)
    return outP.reshape(npad, 8)[:N, :F2]


# speedup vs baseline: 76.2434x; 1.2384x over previous
import functools

import jax
import jax.numpy as jnp
from jax import lax
from jax.experimental import pallas as pl
from jax.experimental.pallas import tpu as pltpu
from jax.experimental.pallas import tpu_sc as plsc

NC = 2
NS = 16
NW = NC * NS
LANE = 128
K = 16
PK = 16


def _sc_hist(col2d, z_tile, npad, wn):
    gpw = wn * K
    rpt = npad // NS

    @functools.partial(
        pl.kernel,
        out_type=jax.ShapeDtypeStruct((NC * npad,), jnp.float32),
        mesh=plsc.VectorSubcoreMesh(core_axis_name="c", subcore_axis_name="s"),
        scratch_types=[
            pltpu.VMEM_SHARED((npad,), jnp.float32),
            pltpu.VMEM((K, LANE), jnp.int32),
            pltpu.VMEM((LANE,), jnp.float32),
            pltpu.VMEM((rpt,), jnp.float32),
            pltpu.SemaphoreType.DMA,
        ],
        compiler_params=pltpu.CompilerParams(use_tc_tiling_on_sc=False),
    )
    def hist(col_hbm, z_hbm, out_hbm, acc, col_buf, ones, zbuf, sem):
        cid = lax.axis_index("c")
        sid = lax.axis_index("s")
        w = cid * NS + sid
        pltpu.sync_copy(z_hbm, zbuf)
        pltpu.sync_copy(zbuf, acc.at[pl.ds(sid * rpt, rpt)])
        for t in range(LANE // 16):
            ones[pl.ds(t * 16, 16)] = jnp.ones((16,), jnp.float32)
        plsc.subcore_barrier()

        def body(win, carry):
            g0 = w * gpw + win * K
            pltpu.sync_copy(col_hbm.at[pl.ds(g0, K)], col_buf)
            descs = [
                pltpu.async_copy(ones, acc.at[col_buf.at[j]], sem, add=True)
                for j in range(K)
            ]
            for d in descs:
                d.wait()
            return carry

        lax.fori_loop(0, wn, body, 0)
        plsc.subcore_barrier()
        pltpu.sync_copy(acc.at[pl.ds(sid * rpt, rpt)], zbuf)
        pltpu.sync_copy(zbuf, out_hbm.at[pl.ds(cid * npad + sid * rpt, rpt)])

    return hist(col2d, z_tile)


def _sc_prop(row2d, col2d, y, z_tile, npad, wn, F):
    gpw = wn * K
    rpt = npad // NS

    @functools.partial(
        pl.kernel,
        out_type=jax.ShapeDtypeStruct((NC * npad, F), jnp.float32),
        mesh=plsc.VectorSubcoreMesh(core_axis_name="c", subcore_axis_name="s"),
        scratch_types=[
            pltpu.VMEM_SHARED((npad, F), jnp.float32),
            pltpu.VMEM((K, LANE), jnp.int32),
            pltpu.VMEM((K, LANE), jnp.int32),
            pltpu.VMEM((K * LANE, F), jnp.float32),
            pltpu.VMEM((rpt, F), jnp.float32),
            pltpu.SemaphoreType.DMA,
            pltpu.SemaphoreType.DMA,
        ],
        compiler_params=pltpu.CompilerParams(use_tc_tiling_on_sc=False),
    )
    def prop(row_hbm, col_hbm, y_hbm, z_hbm, out_hbm,
             acc, row_buf, col_buf, gbuf, zbuf, gsem, ssem):
        cid = lax.axis_index("c")
        sid = lax.axis_index("s")
        w = cid * NS + sid
        pltpu.sync_copy(z_hbm, zbuf)
        pltpu.sync_copy(zbuf, acc.at[pl.ds(sid * rpt, rpt)])
        plsc.subcore_barrier()

        def body(win, carry):
            g0 = w * gpw + win * K
            pltpu.sync_copy(row_hbm.at[pl.ds(g0, K)], row_buf)
            pltpu.sync_copy(col_hbm.at[pl.ds(g0, K)], col_buf)
            gd = [
                pltpu.async_copy(y_hbm.at[row_buf.at[j]],
                                 gbuf.at[pl.ds(j * LANE, LANE)], gsem)
                for j in range(K)
            ]
            sd = []
            for j in range(K):
                gd[j].wait()
                sd.append(pltpu.async_copy(gbuf.at[pl.ds(j * LANE, LANE)],
                                           acc.at[col_buf.at[j]], ssem,
                                           add=True))
            for d in sd:
                d.wait()
            return carry

        lax.fori_loop(0, wn, body, 0)
        plsc.subcore_barrier()
        pltpu.sync_copy(acc.at[pl.ds(sid * rpt, rpt)], zbuf)
        pltpu.sync_copy(zbuf, out_hbm.at[pl.ds(cid * npad + sid * rpt, rpt)])

    return prop(row2d, col2d, y, z_tile)


_TCB = 800
_BQ = 200


def _tc_pack(x_r, degp3, npad):
    nq = npad // 128

    def body(x_ref, dp_ref, xp_ref, dis_ref):
        deg = dp_ref[0] + dp_ref[1] + 1.0
        dis = lax.rsqrt(deg)
        lane = lax.broadcasted_iota(jnp.int32, (_BQ, 128), 1)
        for s in range(8):
            dis_ref[:, s, :] = jnp.take_along_axis(dis, 16 * s + lane // 8,
                                                   axis=1)
        xq = x_ref[...]
        lane2 = lax.broadcasted_iota(jnp.int32, (4 * _BQ, 128), 1)
        j = lane2 // 8
        k = lane2 % 8
        for h in range(2):
            idx = jnp.minimum(64 * h + 4 * j + k, 127)
            v = jnp.take_along_axis(xq, idx, axis=1)
            xp_ref[:, h, :] = jnp.where(k < 4, v, 0.0)

    return pl.pallas_call(
        body,
        grid=(nq // _BQ,),
        in_specs=[
            pl.BlockSpec((4 * _BQ, 128), lambda i: (i, 0)),
            pl.BlockSpec((NC, _BQ, 128), lambda i: (0, i, 0)),
        ],
        out_specs=[
            pl.BlockSpec((4 * _BQ, 2, 128), lambda i: (i, 0, 0)),
            pl.BlockSpec((_BQ, 8, 128), lambda i: (i, 0, 0)),
        ],
        out_shape=[
            jax.ShapeDtypeStruct((npad // 32, 2, 128), jnp.float32),
            jax.ShapeDtypeStruct((nq, 8, 128), jnp.float32),
        ],
    )(x_r, degp3)


def _tc_pre(xP, W1B, disP, npadP):
    def body(x_ref, w_ref, dis_ref, y1_ref):
        xw = jnp.dot(x_ref[...], w_ref[...], preferred_element_type=jnp.float32)
        y1_ref[...] = xw * dis_ref[...]

    return pl.pallas_call(
        body,
        grid=(npadP // _TCB,),
        in_specs=[
            pl.BlockSpec((_TCB, 128), lambda i: (i, 0)),
            pl.BlockSpec((128, 128), lambda i: (0, 0)),
            pl.BlockSpec((_TCB, 128), lambda i: (i, 0)),
        ],
        out_specs=pl.BlockSpec((_TCB, 128), lambda i: (i, 0)),
        out_shape=jax.ShapeDtypeStruct((npadP, 128), jnp.float32),
    )(xP, W1B, disP)


def _tc_mid(acc1P, y1P, disP, W2B, b1P, npadP):
    def body(a_ref, y1_ref, dis_ref, w_ref, b_ref, y2_ref):
        dis = dis_ref[...]
        h = jnp.maximum(dis * (a_ref[0] + a_ref[1] + y1_ref[...]) + b_ref[...],
                        0.0)
        y2_ref[...] = jnp.dot(h, w_ref[...],
                              preferred_element_type=jnp.float32) * dis

    return pl.pallas_call(
        body,
        grid=(npadP // _TCB,),
        in_specs=[
            pl.BlockSpec((NC, _TCB, 128), lambda i: (0, i, 0)),
            pl.BlockSpec((_TCB, 128), lambda i: (i, 0)),
            pl.BlockSpec((_TCB, 128), lambda i: (i, 0)),
            pl.BlockSpec((128, 128), lambda i: (0, 0)),
            pl.BlockSpec((1, 128), lambda i: (0, 0)),
        ],
        out_specs=pl.BlockSpec((_TCB, 128), lambda i: (i, 0)),
        out_shape=jax.ShapeDtypeStruct((npadP, 128), jnp.float32),
    )(acc1P, y1P, disP, W2B, b1P)


def _tc_post(acc2P, y2P, disP, b2P, npadP):
    def body(a_ref, y2_ref, dis_ref, b_ref, o_ref):
        o_ref[...] = (dis_ref[...] * (a_ref[0] + a_ref[1] + y2_ref[...])
                      + b_ref[...])

    return pl.pallas_call(
        body,
        grid=(npadP // _TCB,),
        in_specs=[
            pl.BlockSpec((NC, _TCB, 128), lambda i: (0, i, 0)),
            pl.BlockSpec((_TCB, 128), lambda i: (i, 0)),
            pl.BlockSpec((_TCB, 128), lambda i: (i, 0)),
            pl.BlockSpec((1, 128), lambda i: (0, 0)),
        ],
        out_specs=pl.BlockSpec((_TCB, 128), lambda i: (i, 0)),
        out_shape=jax.ShapeDtypeStruct((npadP, 128), jnp.float32),
    )(acc2P, y2P, disP, b2P)


def kernel(x, edge_index, W1, b1, W2, b2):
    N = x.shape[0]
    E = edge_index.shape[1]
    npad = 102400 if N == 100000 else ((N + 64 + 16 * _TCB - 1)
                                       // (16 * _TCB)) * (16 * _TCB)
    npadP = npad // PK

    G = -(-E // LANE)
    wn = -(-G // (NW * K))
    Gp = NW * K * wn
    pad = Gp * LANE - E

    row = edge_index[0].astype(jnp.int32)
    col = edge_index[1].astype(jnp.int32)
    ar = jnp.arange(pad, dtype=jnp.int32)
    row2d = jnp.concatenate([row, ar % 64]).reshape(Gp, LANE)
    col2d = jnp.concatenate([col, N + (ar % 64)]).reshape(Gp, LANE)

    rpt = npad // NS
    z1 = jnp.zeros((rpt,), jnp.float32)
    z8 = jnp.zeros((rpt, 8), jnp.float32)

    x_r = jnp.pad(x.reshape(N * x.shape[1] // 128, 128),
                  ((0, (npad - N) * x.shape[1] // 128), (0, 0)))
    W1B = jnp.kron(jnp.eye(PK, dtype=jnp.float32),
                   jnp.pad(W1, ((0, 8 - W1.shape[0]), (0, 8 - W1.shape[1]))))
    F2 = W2.shape[1]
    W2B = jnp.kron(jnp.eye(PK, dtype=jnp.float32),
                   jnp.pad(W2, ((0, 8 - W2.shape[0]), (0, 8 - F2))))
    b1P = jnp.tile(jnp.pad(b1, (0, 8 - b1.shape[0])), PK).reshape(1, 128)
    b2P = jnp.tile(jnp.pad(b2, (0, 8 - F2)), PK).reshape(1, 128)

    degp = _sc_hist(col2d, z1, npad, wn)
    xP3, disP3 = _tc_pack(x_r, degp.reshape(NC, npad // 128, 128), npad)
    xP = xP3.reshape(npadP, 128)
    disP = disP3.reshape(npadP, 128)
    y1P = _tc_pre(xP, W1B, disP, npadP)
    acc1 = _sc_prop(row2d, col2d, y1P.reshape(npad, 8), z8, npad, wn, 8)
    y2P = _tc_mid(acc1.reshape(NC, npadP, 128), y1P, disP, W2B, b1P, npadP)
    acc2 = _sc_prop(row2d, col2d, y2P.reshape(npad, 8), z8, npad, wn, 8)
    outP = _tc_post(acc2.reshape(NC, npadP, 128), y2P, disP, b2P, npadP)
    return outP.reshape(npad, 8)[:N, :F2]


# speedup vs baseline: 85.9175x; 1.1269x over previous
import functools

import jax
import jax.numpy as jnp
from jax import lax
from jax.experimental import pallas as pl
from jax.experimental.pallas import tpu as pltpu
from jax.experimental.pallas import tpu_sc as plsc

NC = 2
NS = 16
NW = NC * NS
LANE = 128
K = 8
PK = 16


def _sc_hist(col2d, z_tile, npad, wn):
    gpw = wn * K
    rpt = npad // NS

    @functools.partial(
        pl.kernel,
        out_type=jax.ShapeDtypeStruct((NC * npad,), jnp.float32),
        mesh=plsc.VectorSubcoreMesh(core_axis_name="c", subcore_axis_name="s"),
        scratch_types=[
            pltpu.VMEM_SHARED((npad,), jnp.float32),
            pltpu.VMEM((2, K, LANE), jnp.int32),
            pltpu.VMEM((LANE,), jnp.float32),
            pltpu.VMEM((rpt,), jnp.float32),
            pltpu.SemaphoreType.DMA,
            pltpu.SemaphoreType.DMA,
        ],
        compiler_params=pltpu.CompilerParams(use_tc_tiling_on_sc=False),
    )
    def hist(col_hbm, z_hbm, out_hbm, acc, col_buf, ones, zbuf, sem, isem):
        cid = lax.axis_index("c")
        sid = lax.axis_index("s")
        w = cid * NS + sid
        base = w * gpw
        pltpu.sync_copy(z_hbm, zbuf)
        pltpu.sync_copy(zbuf, acc.at[pl.ds(sid * rpt, rpt)])
        for t in range(LANE // 16):
            ones[pl.ds(t * 16, 16)] = jnp.ones((16,), jnp.float32)
        pltpu.sync_copy(col_hbm.at[pl.ds(base, K)], col_buf.at[0])
        plsc.subcore_barrier()

        def body(win, carry):
            p = win & 1

            @pl.when(win != 0)
            def _():
                pltpu.make_async_copy(col_hbm.at[pl.ds(base, K)],
                                      col_buf.at[p], isem).wait()
                for _j in range(K):
                    pltpu.make_async_copy(z_hbm.at[pl.ds(0, LANE)],
                                          ones, sem).wait()

            @pl.when(win + 1 != wn)
            def _():
                pltpu.async_copy(col_hbm.at[pl.ds(base + (win + 1) * K, K)],
                                 col_buf.at[1 - p], isem)

            for j in range(K):
                pltpu.async_copy(ones, acc.at[col_buf.at[p].at[j]], sem,
                                 add=True)
            return carry

        lax.fori_loop(0, wn, body, 0)
        for _j in range(K):
            pltpu.make_async_copy(z_hbm.at[pl.ds(0, LANE)], ones, sem).wait()
        plsc.subcore_barrier()
        pltpu.sync_copy(acc.at[pl.ds(sid * rpt, rpt)], zbuf)
        pltpu.sync_copy(zbuf, out_hbm.at[pl.ds(cid * npad + sid * rpt, rpt)])

    return hist(col2d, z_tile)


def _sc_prop(row2d, col2d, y, z_tile, npad, wn, F):
    gpw = wn * K
    rpt = npad // NS

    @functools.partial(
        pl.kernel,
        out_type=jax.ShapeDtypeStruct((NC * npad, F), jnp.float32),
        mesh=plsc.VectorSubcoreMesh(core_axis_name="c", subcore_axis_name="s"),
        scratch_types=[
            pltpu.VMEM_SHARED((npad, F), jnp.float32),
            pltpu.VMEM((2, K, LANE), jnp.int32),
            pltpu.VMEM((2, K, LANE), jnp.int32),
            pltpu.VMEM((2, K * LANE, F), jnp.float32),
            pltpu.VMEM((rpt // 2, F), jnp.float32),
            pltpu.SemaphoreType.DMA,
            pltpu.SemaphoreType.DMA,
            pltpu.SemaphoreType.DMA,
        ],
        compiler_params=pltpu.CompilerParams(use_tc_tiling_on_sc=False),
    )
    def prop(row_hbm, col_hbm, y_hbm, z_hbm, out_hbm,
             acc, row_buf, col_buf, gbuf, zbuf, gsem, ssem, isem):
        cid = lax.axis_index("c")
        sid = lax.axis_index("s")
        w = cid * NS + sid
        base = w * gpw
        pltpu.sync_copy(z_hbm, zbuf)
        for c in range(2):
            pltpu.sync_copy(zbuf,
                            acc.at[pl.ds(sid * rpt + c * (rpt // 2), rpt // 2)])
        pltpu.sync_copy(row_hbm.at[pl.ds(base, K)], row_buf.at[0])
        pltpu.sync_copy(col_hbm.at[pl.ds(base, K)], col_buf.at[0])
        plsc.subcore_barrier()

        def body(win, carry):
            p = win & 1

            @pl.when(win != 0)
            def _():
                pltpu.make_async_copy(row_hbm.at[pl.ds(base, K)],
                                      row_buf.at[p], isem).wait()
                pltpu.make_async_copy(col_hbm.at[pl.ds(base, K)],
                                      col_buf.at[p], isem).wait()

            gd = [
                pltpu.async_copy(y_hbm.at[row_buf.at[p].at[j]],
                                 gbuf.at[p].at[pl.ds(j * LANE, LANE)], gsem)
                for j in range(K)
            ]

            @pl.when(win != 0)
            def _():
                pltpu.make_async_copy(y_hbm.at[pl.ds(0, K * LANE)],
                                      gbuf.at[1 - p], ssem).wait()

            @pl.when(win + 1 != wn)
            def _():
                g0n = base + (win + 1) * K
                pltpu.async_copy(row_hbm.at[pl.ds(g0n, K)],
                                 row_buf.at[1 - p], isem)
                pltpu.async_copy(col_hbm.at[pl.ds(g0n, K)],
                                 col_buf.at[1 - p], isem)

            for j in range(K):
                gd[j].wait()
                pltpu.async_copy(gbuf.at[p].at[pl.ds(j * LANE, LANE)],
                                 acc.at[col_buf.at[p].at[j]], ssem, add=True)
            return carry

        lax.fori_loop(0, wn, body, 0)
        pltpu.make_async_copy(y_hbm.at[pl.ds(0, K * LANE)],
                              gbuf.at[(wn - 1) & 1], ssem).wait()
        plsc.subcore_barrier()
        for c in range(2):
            o0 = sid * rpt + c * (rpt // 2)
            pltpu.sync_copy(acc.at[pl.ds(o0, rpt // 2)], zbuf)
            pltpu.sync_copy(zbuf, out_hbm.at[pl.ds(cid * npad + o0, rpt // 2)])

    return prop(row2d, col2d, y, z_tile)


_TCB = 800
_BQ = 200


def _tc_pack(x_r, degp3, npad):
    nq = npad // 128

    def body(x_ref, dp_ref, xp_ref, dis_ref):
        deg = dp_ref[0] + dp_ref[1] + 1.0
        dis = lax.rsqrt(deg)
        lane = lax.broadcasted_iota(jnp.int32, (_BQ, 128), 1)
        for s in range(8):
            dis_ref[:, s, :] = jnp.take_along_axis(dis, 16 * s + lane // 8,
                                                   axis=1)
        xq = x_ref[...]
        lane2 = lax.broadcasted_iota(jnp.int32, (4 * _BQ, 128), 1)
        j = lane2 // 8
        k = lane2 % 8
        for h in range(2):
            idx = jnp.minimum(64 * h + 4 * j + k, 127)
            v = jnp.take_along_axis(xq, idx, axis=1)
            xp_ref[:, h, :] = jnp.where(k < 4, v, 0.0)

    return pl.pallas_call(
        body,
        grid=(nq // _BQ,),
        in_specs=[
            pl.BlockSpec((4 * _BQ, 128), lambda i: (i, 0)),
            pl.BlockSpec((NC, _BQ, 128), lambda i: (0, i, 0)),
        ],
        out_specs=[
            pl.BlockSpec((4 * _BQ, 2, 128), lambda i: (i, 0, 0)),
            pl.BlockSpec((_BQ, 8, 128), lambda i: (i, 0, 0)),
        ],
        out_shape=[
            jax.ShapeDtypeStruct((npad // 32, 2, 128), jnp.float32),
            jax.ShapeDtypeStruct((nq, 8, 128), jnp.float32),
        ],
    )(x_r, degp3)


def _tc_pre(xP, W1B, disP, npadP):
    def body(x_ref, w_ref, dis_ref, y1_ref):
        xw = jnp.dot(x_ref[...], w_ref[...], preferred_element_type=jnp.float32)
        y1_ref[...] = xw * dis_ref[...]

    return pl.pallas_call(
        body,
        grid=(npadP // _TCB,),
        in_specs=[
            pl.BlockSpec((_TCB, 128), lambda i: (i, 0)),
            pl.BlockSpec((128, 128), lambda i: (0, 0)),
            pl.BlockSpec((_TCB, 128), lambda i: (i, 0)),
        ],
        out_specs=pl.BlockSpec((_TCB, 128), lambda i: (i, 0)),
        out_shape=jax.ShapeDtypeStruct((npadP, 128), jnp.float32),
    )(xP, W1B, disP)


def _tc_mid(acc1P, y1P, disP, W2B, b1P, npadP):
    def body(a_ref, y1_ref, dis_ref, w_ref, b_ref, y2_ref):
        dis = dis_ref[...]
        h = jnp.maximum(dis * (a_ref[0] + a_ref[1] + y1_ref[...]) + b_ref[...],
                        0.0)
        y2_ref[...] = jnp.dot(h, w_ref[...],
                              preferred_element_type=jnp.float32) * dis

    return pl.pallas_call(
        body,
        grid=(npadP // _TCB,),
        in_specs=[
            pl.BlockSpec((NC, _TCB, 128), lambda i: (0, i, 0)),
            pl.BlockSpec((_TCB, 128), lambda i: (i, 0)),
            pl.BlockSpec((_TCB, 128), lambda i: (i, 0)),
            pl.BlockSpec((128, 128), lambda i: (0, 0)),
            pl.BlockSpec((1, 128), lambda i: (0, 0)),
        ],
        out_specs=pl.BlockSpec((_TCB, 128), lambda i: (i, 0)),
        out_shape=jax.ShapeDtypeStruct((npadP, 128), jnp.float32),
    )(acc1P, y1P, disP, W2B, b1P)


def _tc_post(acc2P, y2P, disP, b2P, npadP):
    def body(a_ref, y2_ref, dis_ref, b_ref, o_ref):
        o_ref[...] = (dis_ref[...] * (a_ref[0] + a_ref[1] + y2_ref[...])
                      + b_ref[...])

    return pl.pallas_call(
        body,
        grid=(npadP // _TCB,),
        in_specs=[
            pl.BlockSpec((NC, _TCB, 128), lambda i: (0, i, 0)),
            pl.BlockSpec((_TCB, 128), lambda i: (i, 0)),
            pl.BlockSpec((_TCB, 128), lambda i: (i, 0)),
            pl.BlockSpec((1, 128), lambda i: (0, 0)),
        ],
        out_specs=pl.BlockSpec((_TCB, 128), lambda i: (i, 0)),
        out_shape=jax.ShapeDtypeStruct((npadP, 128), jnp.float32),
    )(acc2P, y2P, disP, b2P)


def kernel(x, edge_index, W1, b1, W2, b2):
    N = x.shape[0]
    E = edge_index.shape[1]
    npad = 102400 if N == 100000 else ((N + 64 + 16 * _TCB - 1)
                                       // (16 * _TCB)) * (16 * _TCB)
    npadP = npad // PK

    G = -(-E // LANE)
    wn = -(-G // (NW * K))
    Gp = NW * K * wn
    pad = Gp * LANE - E

    row = edge_index[0].astype(jnp.int32)
    col = edge_index[1].astype(jnp.int32)
    ar = jnp.arange(pad, dtype=jnp.int32)
    row2d = jnp.concatenate([row, ar % 64]).reshape(Gp, LANE)
    col2d = jnp.concatenate([col, N + (ar % 64)]).reshape(Gp, LANE)

    rpt = npad // NS
    z1 = jnp.zeros((rpt,), jnp.float32)
    z8 = jnp.zeros((rpt // 2, 8), jnp.float32)

    x_r = jnp.pad(x.reshape(N * x.shape[1] // 128, 128),
                  ((0, (npad - N) * x.shape[1] // 128), (0, 0)))
    W1B = jnp.kron(jnp.eye(PK, dtype=jnp.float32),
                   jnp.pad(W1, ((0, 8 - W1.shape[0]), (0, 8 - W1.shape[1]))))
    F2 = W2.shape[1]
    W2B = jnp.kron(jnp.eye(PK, dtype=jnp.float32),
                   jnp.pad(W2, ((0, 8 - W2.shape[0]), (0, 8 - F2))))
    b1P = jnp.tile(jnp.pad(b1, (0, 8 - b1.shape[0])), PK).reshape(1, 128)
    b2P = jnp.tile(jnp.pad(b2, (0, 8 - F2)), PK).reshape(1, 128)

    degp = _sc_hist(col2d, z1, npad, wn)
    xP3, disP3 = _tc_pack(x_r, degp.reshape(NC, npad // 128, 128), npad)
    xP = xP3.reshape(npadP, 128)
    disP = disP3.reshape(npadP, 128)
    y1P = _tc_pre(xP, W1B, disP, npadP)
    acc1 = _sc_prop(row2d, col2d, y1P.reshape(npad, 8), z8, npad, wn, 8)
    y2P = _tc_mid(acc1.reshape(NC, npadP, 128), y1P, disP, W2B, b1P, npadP)
    acc2 = _sc_prop(row2d, col2d, y2P.reshape(npad, 8), z8, npad, wn, 8)
    outP = _tc_post(acc2.reshape(NC, npadP, 128), y2P, disP, b2P, npadP)
    return outP.reshape(npad, 8)[:N, :F2]


# speedup vs baseline: 95.2403x; 1.1085x over previous
import functools

import jax
import jax.numpy as jnp
from jax import lax
from jax.experimental import pallas as pl
from jax.experimental.pallas import tpu as pltpu
from jax.experimental.pallas import tpu_sc as plsc

NC = 2
NS = 16
NW = NC * NS
LANE = 128
K = 8
PK = 16


def _sc_hist(col2d, z_tile, npad, wn):
    gpw = wn * K
    rpt = npad // NS

    @functools.partial(
        pl.kernel,
        out_type=jax.ShapeDtypeStruct((NC * npad,), jnp.float32),
        mesh=plsc.VectorSubcoreMesh(core_axis_name="c", subcore_axis_name="s"),
        scratch_types=[
            pltpu.VMEM_SHARED((npad,), jnp.float32),
            pltpu.VMEM((2, K, LANE), jnp.int32),
            pltpu.VMEM((LANE,), jnp.float32),
            pltpu.VMEM((rpt,), jnp.float32),
            pltpu.SemaphoreType.DMA,
            pltpu.SemaphoreType.DMA,
        ],
        compiler_params=pltpu.CompilerParams(use_tc_tiling_on_sc=False),
    )
    def hist(col_hbm, z_hbm, out_hbm, acc, col_buf, ones, zbuf, sem, isem):
        cid = lax.axis_index("c")
        sid = lax.axis_index("s")
        w = cid * NS + sid
        base = w * gpw
        pltpu.sync_copy(z_hbm, zbuf)
        pltpu.sync_copy(zbuf, acc.at[pl.ds(sid * rpt, rpt)])
        for t in range(LANE // 16):
            ones[pl.ds(t * 16, 16)] = jnp.ones((16,), jnp.float32)
        pltpu.sync_copy(col_hbm.at[pl.ds(base, K)], col_buf.at[0])
        plsc.subcore_barrier()

        def body(win, carry):
            p = win & 1

            @pl.when(win != 0)
            def _():
                pltpu.make_async_copy(col_hbm.at[pl.ds(base, K)],
                                      col_buf.at[p], isem).wait()
                for _j in range(K):
                    pltpu.make_async_copy(z_hbm.at[pl.ds(0, LANE)],
                                          ones, sem).wait()

            @pl.when(win + 1 != wn)
            def _():
                pltpu.async_copy(col_hbm.at[pl.ds(base + (win + 1) * K, K)],
                                 col_buf.at[1 - p], isem)

            for j in range(K):
                pltpu.async_copy(ones, acc.at[col_buf.at[p].at[j]], sem,
                                 add=True)
            return carry

        lax.fori_loop(0, wn, body, 0)
        for _j in range(K):
            pltpu.make_async_copy(z_hbm.at[pl.ds(0, LANE)], ones, sem).wait()
        plsc.subcore_barrier()
        pltpu.sync_copy(acc.at[pl.ds(sid * rpt, rpt)], zbuf)
        pltpu.sync_copy(zbuf, out_hbm.at[pl.ds(cid * npad + sid * rpt, rpt)])

    return hist(col2d, z_tile)


def _sc_prop(row2d, col2d, y, z_tile, npad, wn, F):
    gpw = wn * K
    rpt = npad // NS

    @functools.partial(
        pl.kernel,
        out_type=jax.ShapeDtypeStruct((NC * npad, F), jnp.float32),
        mesh=plsc.VectorSubcoreMesh(core_axis_name="c", subcore_axis_name="s"),
        scratch_types=[
            pltpu.VMEM_SHARED((npad, F), jnp.float32),
            pltpu.VMEM((2, K, LANE), jnp.int32),
            pltpu.VMEM((2, K, LANE), jnp.int32),
            pltpu.VMEM((2, K * LANE, F), jnp.float32),
            pltpu.VMEM((rpt // 2, F), jnp.float32),
            pltpu.SemaphoreType.DMA,
            pltpu.SemaphoreType.DMA,
            pltpu.SemaphoreType.DMA,
        ],
        compiler_params=pltpu.CompilerParams(use_tc_tiling_on_sc=False),
    )
    def prop(row_hbm, col_hbm, y_hbm, z_hbm, out_hbm,
             acc, row_buf, col_buf, gbuf, zbuf, gsem, ssem, isem):
        cid = lax.axis_index("c")
        sid = lax.axis_index("s")
        w = cid * NS + sid
        base = w * gpw
        pltpu.sync_copy(z_hbm, zbuf)
        for c in range(2):
            pltpu.sync_copy(zbuf,
                            acc.at[pl.ds(sid * rpt + c * (rpt // 2), rpt // 2)])
        pltpu.sync_copy(row_hbm.at[pl.ds(base, K)], row_buf.at[0])
        pltpu.sync_copy(col_hbm.at[pl.ds(base, K)], col_buf.at[0])
        plsc.subcore_barrier()

        def body(win, carry):
            p = win & 1

            @pl.when(win != 0)
            def _():
                pltpu.make_async_copy(row_hbm.at[pl.ds(base, K)],
                                      row_buf.at[p], isem).wait()
                pltpu.make_async_copy(col_hbm.at[pl.ds(base, K)],
                                      col_buf.at[p], isem).wait()

            gd = [
                pltpu.async_copy(y_hbm.at[row_buf.at[p].at[j]],
                                 gbuf.at[p].at[pl.ds(j * LANE, LANE)], gsem)
                for j in range(K)
            ]

            @pl.when(win != 0)
            def _():
                pltpu.make_async_copy(y_hbm.at[pl.ds(0, K * LANE)],
                                      gbuf.at[1 - p], ssem).wait()

            @pl.when(win + 1 != wn)
            def _():
                g0n = base + (win + 1) * K
                pltpu.async_copy(row_hbm.at[pl.ds(g0n, K)],
                                 row_buf.at[1 - p], isem)
                pltpu.async_copy(col_hbm.at[pl.ds(g0n, K)],
                                 col_buf.at[1 - p], isem)

            for j in range(K):
                gd[j].wait()
                pltpu.async_copy(gbuf.at[p].at[pl.ds(j * LANE, LANE)],
                                 acc.at[col_buf.at[p].at[j]], ssem, add=True)
            return carry

        lax.fori_loop(0, wn, body, 0)
        pltpu.make_async_copy(y_hbm.at[pl.ds(0, K * LANE)],
                              gbuf.at[(wn - 1) & 1], ssem).wait()
        plsc.subcore_barrier()
        for c in range(2):
            o0 = sid * rpt + c * (rpt // 2)
            pltpu.sync_copy(acc.at[pl.ds(o0, rpt // 2)], zbuf)
            pltpu.sync_copy(zbuf, out_hbm.at[pl.ds(cid * npad + o0, rpt // 2)])

    return prop(row2d, col2d, y, z_tile)


_TCB = 800
_BQ = 200


def _tc_pack(x_r, degp3, npad):
    nq = npad // 128

    def body(x_ref, dp_ref, xp_ref, dis_ref):
        deg = dp_ref[0] + dp_ref[1] + 1.0
        dis = lax.rsqrt(deg)
        lane = lax.broadcasted_iota(jnp.int32, (_BQ, 128), 1)
        for s in range(8):
            dis_ref[:, s, :] = jnp.take_along_axis(dis, 16 * s + lane // 8,
                                                   axis=1)
        xq = x_ref[...]
        lane2 = lax.broadcasted_iota(jnp.int32, (4 * _BQ, 128), 1)
        j = lane2 // 8
        k = lane2 % 8
        for h in range(2):
            idx = jnp.minimum(64 * h + 4 * j + k, 127)
            v = jnp.take_along_axis(xq, idx, axis=1)
            xp_ref[:, h, :] = jnp.where(k < 4, v, 0.0)

    return pl.pallas_call(
        body,
        grid=(nq // _BQ,),
        in_specs=[
            pl.BlockSpec((4 * _BQ, 128), lambda i: (i, 0)),
            pl.BlockSpec((NC, _BQ, 128), lambda i: (0, i, 0)),
        ],
        out_specs=[
            pl.BlockSpec((4 * _BQ, 2, 128), lambda i: (i, 0, 0)),
            pl.BlockSpec((_BQ, 8, 128), lambda i: (i, 0, 0)),
        ],
        out_shape=[
            jax.ShapeDtypeStruct((npad // 32, 2, 128), jnp.float32),
            jax.ShapeDtypeStruct((nq, 8, 128), jnp.float32),
        ],
    )(x_r, degp3)


def _tc_pre(xP, W1B, disP, npadP):
    def body(x_ref, w_ref, dis_ref, y1_ref):
        xw = jnp.dot(x_ref[...], w_ref[...], preferred_element_type=jnp.float32)
        y1_ref[...] = xw * dis_ref[...]

    return pl.pallas_call(
        body,
        grid=(npadP // _TCB,),
        in_specs=[
            pl.BlockSpec((_TCB, 128), lambda i: (i, 0)),
            pl.BlockSpec((128, 128), lambda i: (0, 0)),
            pl.BlockSpec((_TCB, 128), lambda i: (i, 0)),
        ],
        out_specs=pl.BlockSpec((_TCB, 128), lambda i: (i, 0)),
        out_shape=jax.ShapeDtypeStruct((npadP, 128), jnp.float32),
    )(xP, W1B, disP)


def _tc_mid(acc1P, y1P, disP, W2B, b1P, npadP):
    def body(a_ref, y1_ref, dis_ref, w_ref, b_ref, y2_ref):
        dis = dis_ref[...]
        h = jnp.maximum(dis * (a_ref[0] + a_ref[1] + y1_ref[...]) + b_ref[...],
                        0.0)
        y2_ref[...] = jnp.dot(h, w_ref[...],
                              preferred_element_type=jnp.float32) * dis

    return pl.pallas_call(
        body,
        grid=(npadP // _TCB,),
        in_specs=[
            pl.BlockSpec((NC, _TCB, 128), lambda i: (0, i, 0)),
            pl.BlockSpec((_TCB, 128), lambda i: (i, 0)),
            pl.BlockSpec((_TCB, 128), lambda i: (i, 0)),
            pl.BlockSpec((128, 128), lambda i: (0, 0)),
            pl.BlockSpec((1, 128), lambda i: (0, 0)),
        ],
        out_specs=pl.BlockSpec((_TCB, 128), lambda i: (i, 0)),
        out_shape=jax.ShapeDtypeStruct((npadP, 128), jnp.float32),
    )(acc1P, y1P, disP, W2B, b1P)


def _tc_post(acc2P, y2P, disP, b2P, npadP):
    def body(a_ref, y2_ref, dis_ref, b_ref, o_ref):
        o_ref[...] = (dis_ref[...] * (a_ref[0] + a_ref[1] + y2_ref[...])
                      + b_ref[...])

    return pl.pallas_call(
        body,
        grid=(npadP // _TCB,),
        in_specs=[
            pl.BlockSpec((NC, _TCB, 128), lambda i: (0, i, 0)),
            pl.BlockSpec((_TCB, 128), lambda i: (i, 0)),
            pl.BlockSpec((_TCB, 128), lambda i: (i, 0)),
            pl.BlockSpec((1, 128), lambda i: (0, 0)),
        ],
        out_specs=pl.BlockSpec((_TCB, 128), lambda i: (i, 0)),
        out_shape=jax.ShapeDtypeStruct((npadP, 128), jnp.float32),
    )(acc2P, y2P, disP, b2P)


def kernel(x, edge_index, W1, b1, W2, b2):
    N = x.shape[0]
    E = edge_index.shape[1]
    npad = 102400 if N == 100000 else ((N + 64 + 16 * _TCB - 1)
                                       // (16 * _TCB)) * (16 * _TCB)
    npadP = npad // PK

    G = -(-E // LANE)
    wn = -(-G // (NW * K))
    Gp = NW * K * wn
    pad = Gp * LANE - E

    row = edge_index[0].astype(jnp.int32)
    col = edge_index[1].astype(jnp.int32)
    ar = jnp.arange(pad, dtype=jnp.int32)
    row2d = jnp.concatenate([row, ar % 64]).reshape(Gp, LANE)
    col2d = jnp.concatenate([col, N + (ar % 64)]).reshape(Gp, LANE)

    rpt = npad // NS
    z1 = jnp.zeros((rpt,), jnp.float32)
    z8 = jnp.zeros((rpt // 2, 8), jnp.float32)

    x_r = jnp.pad(x.reshape(N * x.shape[1] // 128, 128),
                  ((0, (npad - N) * x.shape[1] // 128), (0, 0)))
    W1B = jnp.kron(jnp.eye(PK, dtype=jnp.float32),
                   jnp.pad(W1, ((0, 8 - W1.shape[0]), (0, 8 - W1.shape[1]))))
    F2 = W2.shape[1]
    W2B = jnp.kron(jnp.eye(PK, dtype=jnp.float32),
                   jnp.pad(W2, ((0, 8 - W2.shape[0]), (0, 8 - F2))))
    b1P = jnp.tile(jnp.pad(b1, (0, 8 - b1.shape[0])), PK).reshape(1, 128)
    b2P = jnp.tile(jnp.pad(b2, (0, 8 - F2)), PK).reshape(1, 128)

    degp = _sc_hist(col2d, z1, npad, wn)
    xP3, disP3 = _tc_pack(x_r, degp.reshape(NC, npad // 128, 128), npad)
    xP = xP3.reshape(npadP, 128)
    disP = disP3.reshape(npadP, 128)
    y1P = _tc_pre(xP, W1B, disP, npadP)
    acc1 = _sc_prop(row2d, col2d, y1P.reshape(npad, 8), z8, npad, wn, 8)
    y2P = _tc_mid(acc1.reshape(NC, npadP, 128), y1P, disP, W2B, b1P, npadP)
    acc2 = _sc_prop(row2d, col2d, y2P.reshape(npad, 8), z8, npad, wn, 8)
    outP = _tc_post(acc2.reshape(NC, npadP, 128), y2P, disP, b2P, npadP)
    out1d = outP.reshape(npad * 8)
    cols = [lax.slice(out1d, (k,), (N * 8,), (8,)) for k in range(F2)]
    return jnp.stack(cols, axis=1)


# speedup vs baseline: 113.7057x; 1.1939x over previous
import functools

import jax
import jax.numpy as jnp
from jax import lax
from jax.experimental import pallas as pl
from jax.experimental.pallas import tpu as pltpu
from jax.experimental.pallas import tpu_sc as plsc

NC = 2
NS = 16
NW = NC * NS
LANE = 128
K = 8
PK = 16


def _sc_hist(ei3, aux, z_tile, npad, wn, ag):
    gpw = wn * K
    rpt = npad // NS

    @functools.partial(
        pl.kernel,
        out_type=jax.ShapeDtypeStruct((NC * npad,), jnp.float32),
        mesh=plsc.VectorSubcoreMesh(core_axis_name="c", subcore_axis_name="s"),
        scratch_types=[
            pltpu.VMEM_SHARED((npad,), jnp.float32),
            pltpu.VMEM((2 * K, 2, LANE), jnp.int32),
            pltpu.VMEM((ag, 2, LANE), jnp.int32),
            pltpu.VMEM((LANE,), jnp.float32),
            pltpu.VMEM((rpt,), jnp.float32),
            pltpu.SemaphoreType.DMA,
            pltpu.SemaphoreType.DMA,
        ],
        compiler_params=pltpu.CompilerParams(use_tc_tiling_on_sc=False),
    )
    def hist(ei_hbm, aux_hbm, z_hbm, out_hbm,
             acc, eibuf, abuf, ones, zbuf, sem, isem):
        cid = lax.axis_index("c")
        sid = lax.axis_index("s")
        w = cid * NS + sid
        base = w * gpw
        pltpu.sync_copy(z_hbm, zbuf)
        pltpu.sync_copy(zbuf, acc.at[pl.ds(sid * rpt, rpt)])
        for t in range(LANE // 16):
            ones[pl.ds(t * 16, 16)] = jnp.ones((16,), jnp.float32)
        pltpu.sync_copy(ei_hbm.at[pl.ds(base, K)], eibuf.at[pl.ds(0, K)])
        plsc.subcore_barrier()

        def body(win, carry):
            p = win & 1

            @pl.when(win != 0)
            def _():
                pltpu.make_async_copy(ei_hbm.at[pl.ds(base, K)],
                                      eibuf.at[pl.ds(p * K, K)], isem).wait()
                for _j in range(K):
                    pltpu.make_async_copy(z_hbm.at[pl.ds(0, LANE)],
                                          ones, sem).wait()

            @pl.when(win + 1 != wn)
            def _():
                pltpu.async_copy(ei_hbm.at[pl.ds(base + (win + 1) * K, K)],
                                 eibuf.at[pl.ds((1 - p) * K, K)], isem)

            for j in range(K):
                pltpu.async_copy(ones, acc.at[eibuf.at[p * K + j].at[1]],
                                 sem, add=True)
            return carry

        lax.fori_loop(0, wn, body, 0)
        for _j in range(K):
            pltpu.make_async_copy(z_hbm.at[pl.ds(0, LANE)], ones, sem).wait()
        pltpu.sync_copy(aux_hbm.at[pl.ds(w * ag, ag)], abuf)
        for j in range(ag):
            pltpu.async_copy(ones, acc.at[abuf.at[j].at[1]], sem, add=True)
        for _j in range(ag):
            pltpu.make_async_copy(z_hbm.at[pl.ds(0, LANE)], ones, sem).wait()
        plsc.subcore_barrier()
        pltpu.sync_copy(acc.at[pl.ds(sid * rpt, rpt)], zbuf)
        pltpu.sync_copy(zbuf, out_hbm.at[pl.ds(cid * npad + sid * rpt, rpt)])

    return hist(ei3, aux, z_tile)


def _sc_prop(ei3, aux, y, z_tile, npad, wn, ag, F):
    gpw = wn * K
    rpt = npad // NS

    @functools.partial(
        pl.kernel,
        out_type=jax.ShapeDtypeStruct((NC * npad, F), jnp.float32),
        mesh=plsc.VectorSubcoreMesh(core_axis_name="c", subcore_axis_name="s"),
        scratch_types=[
            pltpu.VMEM_SHARED((npad, F), jnp.float32),
            pltpu.VMEM((2 * K, 2, LANE), jnp.int32),
            pltpu.VMEM((ag, 2, LANE), jnp.int32),
            pltpu.VMEM((2, K * LANE, F), jnp.float32),
            pltpu.VMEM((rpt // 2, F), jnp.float32),
            pltpu.SemaphoreType.DMA,
            pltpu.SemaphoreType.DMA,
            pltpu.SemaphoreType.DMA,
        ],
        compiler_params=pltpu.CompilerParams(use_tc_tiling_on_sc=False),
    )
    def prop(ei_hbm, aux_hbm, y_hbm, z_hbm, out_hbm,
             acc, eibuf, abuf, gbuf, zbuf, gsem, ssem, isem):
        cid = lax.axis_index("c")
        sid = lax.axis_index("s")
        w = cid * NS + sid
        base = w * gpw
        pltpu.sync_copy(z_hbm, zbuf)
        for c in range(2):
            pltpu.sync_copy(zbuf,
                            acc.at[pl.ds(sid * rpt + c * (rpt // 2), rpt // 2)])
        pltpu.sync_copy(ei_hbm.at[pl.ds(base, K)], eibuf.at[pl.ds(0, K)])
        plsc.subcore_barrier()

        def body(win, carry):
            p = win & 1

            @pl.when(win != 0)
            def _():
                pltpu.make_async_copy(ei_hbm.at[pl.ds(base, K)],
                                      eibuf.at[pl.ds(p * K, K)], isem).wait()

            gd = [
                pltpu.async_copy(y_hbm.at[eibuf.at[p * K + j].at[0]],
                                 gbuf.at[p].at[pl.ds(j * LANE, LANE)], gsem)
                for j in range(K)
            ]

            @pl.when(win != 0)
            def _():
                pltpu.make_async_copy(y_hbm.at[pl.ds(0, K * LANE)],
                                      gbuf.at[1 - p], ssem).wait()

            @pl.when(win + 1 != wn)
            def _():
                pltpu.async_copy(ei_hbm.at[pl.ds(base + (win + 1) * K, K)],
                                 eibuf.at[pl.ds((1 - p) * K, K)], isem)

            for j in range(K):
                gd[j].wait()
                pltpu.async_copy(gbuf.at[p].at[pl.ds(j * LANE, LANE)],
                                 acc.at[eibuf.at[p * K + j].at[1]], ssem,
                                 add=True)
            return carry

        lax.fori_loop(0, wn, body, 0)
        pltpu.make_async_copy(y_hbm.at[pl.ds(0, K * LANE)],
                              gbuf.at[(wn - 1) & 1], ssem).wait()
        pltpu.sync_copy(aux_hbm.at[pl.ds(w * ag, ag)], abuf)
        ad = [
            pltpu.async_copy(y_hbm.at[abuf.at[j].at[0]],
                             gbuf.at[0].at[pl.ds(j * LANE, LANE)], gsem)
            for j in range(ag)
        ]
        for j in range(ag):
            ad[j].wait()
            pltpu.async_copy(gbuf.at[0].at[pl.ds(j * LANE, LANE)],
                             acc.at[abuf.at[j].at[1]], ssem, add=True)
        pltpu.make_async_copy(y_hbm.at[pl.ds(0, ag * LANE)],
                              gbuf.at[0].at[pl.ds(0, ag * LANE)], ssem).wait()
        plsc.subcore_barrier()
        for c in range(2):
            o0 = sid * rpt + c * (rpt // 2)
            pltpu.sync_copy(acc.at[pl.ds(o0, rpt // 2)], zbuf)
            pltpu.sync_copy(zbuf, out_hbm.at[pl.ds(cid * npad + o0, rpt // 2)])

    return prop(ei3, aux, y, z_tile)


_TCB = 800
_BQ = 200


def _tc_pack(x_r, degp3, npad):
    nq = npad // 128

    def body(x_ref, dp_ref, xp_ref, dis_ref):
        deg = dp_ref[0] + dp_ref[1] + 1.0
        dis = lax.rsqrt(deg)
        lane = lax.broadcasted_iota(jnp.int32, (_BQ, 128), 1)
        for s in range(8):
            dis_ref[:, s, :] = jnp.take_along_axis(dis, 16 * s + lane // 8,
                                                   axis=1)
        xq = x_ref[...]
        lane2 = lax.broadcasted_iota(jnp.int32, (4 * _BQ, 128), 1)
        j = lane2 // 8
        k = lane2 % 8
        for h in range(2):
            idx = jnp.minimum(64 * h + 4 * j + k, 127)
            v = jnp.take_along_axis(xq, idx, axis=1)
            xp_ref[:, h, :] = jnp.where(k < 4, v, 0.0)

    return pl.pallas_call(
        body,
        grid=(nq // _BQ,),
        in_specs=[
            pl.BlockSpec((4 * _BQ, 128), lambda i: (i, 0)),
            pl.BlockSpec((NC, _BQ, 128), lambda i: (0, i, 0)),
        ],
        out_specs=[
            pl.BlockSpec((4 * _BQ, 2, 128), lambda i: (i, 0, 0)),
            pl.BlockSpec((_BQ, 8, 128), lambda i: (i, 0, 0)),
        ],
        out_shape=[
            jax.ShapeDtypeStruct((npad // 32, 2, 128), jnp.float32),
            jax.ShapeDtypeStruct((nq, 8, 128), jnp.float32),
        ],
    )(x_r, degp3)


def _tc_pre(xP, W1B, disP, npadP):
    def body(x_ref, w_ref, dis_ref, y1_ref):
        xw = jnp.dot(x_ref[...], w_ref[...], preferred_element_type=jnp.float32)
        y1_ref[...] = xw * dis_ref[...]

    return pl.pallas_call(
        body,
        grid=(npadP // _TCB,),
        in_specs=[
            pl.BlockSpec((_TCB, 128), lambda i: (i, 0)),
            pl.BlockSpec((128, 128), lambda i: (0, 0)),
            pl.BlockSpec((_TCB, 128), lambda i: (i, 0)),
        ],
        out_specs=pl.BlockSpec((_TCB, 128), lambda i: (i, 0)),
        out_shape=jax.ShapeDtypeStruct((npadP, 128), jnp.float32),
    )(xP, W1B, disP)


def _tc_mid(acc1P, y1P, disP, W2B, b1P, npadP):
    def body(a_ref, y1_ref, dis_ref, w_ref, b_ref, y2_ref):
        dis = dis_ref[...]
        h = jnp.maximum(dis * (a_ref[0] + a_ref[1] + y1_ref[...]) + b_ref[...],
                        0.0)
        y2_ref[...] = jnp.dot(h, w_ref[...],
                              preferred_element_type=jnp.float32) * dis

    return pl.pallas_call(
        body,
        grid=(npadP // _TCB,),
        in_specs=[
            pl.BlockSpec((NC, _TCB, 128), lambda i: (0, i, 0)),
            pl.BlockSpec((_TCB, 128), lambda i: (i, 0)),
            pl.BlockSpec((_TCB, 128), lambda i: (i, 0)),
            pl.BlockSpec((128, 128), lambda i: (0, 0)),
            pl.BlockSpec((1, 128), lambda i: (0, 0)),
        ],
        out_specs=pl.BlockSpec((_TCB, 128), lambda i: (i, 0)),
        out_shape=jax.ShapeDtypeStruct((npadP, 128), jnp.float32),
    )(acc1P, y1P, disP, W2B, b1P)


def _tc_post(acc2P, y2P, disP, b2P, npadP):
    def body(a_ref, y2_ref, dis_ref, b_ref, o_ref):
        o_ref[...] = (dis_ref[...] * (a_ref[0] + a_ref[1] + y2_ref[...])
                      + b_ref[...])

    return pl.pallas_call(
        body,
        grid=(npadP // _TCB,),
        in_specs=[
            pl.BlockSpec((NC, _TCB, 128), lambda i: (0, i, 0)),
            pl.BlockSpec((_TCB, 128), lambda i: (i, 0)),
            pl.BlockSpec((_TCB, 128), lambda i: (i, 0)),
            pl.BlockSpec((1, 128), lambda i: (0, 0)),
        ],
        out_specs=pl.BlockSpec((_TCB, 128), lambda i: (i, 0)),
        out_shape=jax.ShapeDtypeStruct((npadP, 128), jnp.float32),
    )(acc2P, y2P, disP, b2P)


def kernel(x, edge_index, W1, b1, W2, b2):
    N = x.shape[0]
    E = edge_index.shape[1]
    npad = 102400 if N == 100000 else ((N + 64 + 16 * _TCB - 1)
                                       // (16 * _TCB)) * (16 * _TCB)
    npadP = npad // PK

    ei = edge_index.astype(jnp.int32)
    G3 = E // LANE
    tail_e = E - G3 * LANE
    ei3 = jnp.transpose(ei[:, :G3 * LANE].reshape(2, G3, LANE), (1, 0, 2))
    gpw = (G3 // NW) // K * K
    wn = gpw // K
    rem = G3 - NW * gpw
    ag = -(-(rem + (1 if tail_e else 0)) // NW)
    auxg = NW * ag
    ar = jnp.arange((auxg - rem) * LANE, dtype=jnp.int32)
    prow = (ar % 64).reshape(auxg - rem, 1, LANE)
    pcol = (N + (ar % 64)).reshape(auxg - rem, 1, LANE)
    padgrp = jnp.concatenate([prow, pcol], axis=1)
    if tail_e:
        tr = jnp.concatenate([ei[0, G3 * LANE:], (ar % 64)[:LANE - tail_e]])
        tc = jnp.concatenate([ei[1, G3 * LANE:],
                              N + (ar % 64)[:LANE - tail_e]])
        padgrp = jnp.concatenate(
            [jnp.stack([tr, tc])[None], padgrp[1:]], axis=0)
    aux = jnp.concatenate([ei3[NW * gpw:], padgrp], axis=0)

    rpt = npad // NS
    z1 = jnp.zeros((rpt,), jnp.float32)
    z8 = jnp.zeros((rpt // 2, 8), jnp.float32)

    x_r = jnp.pad(x.reshape(N * x.shape[1] // 128, 128),
                  ((0, (npad - N) * x.shape[1] // 128), (0, 0)))
    W1B = jnp.kron(jnp.eye(PK, dtype=jnp.float32),
                   jnp.pad(W1, ((0, 8 - W1.shape[0]), (0, 8 - W1.shape[1]))))
    F2 = W2.shape[1]
    W2B = jnp.kron(jnp.eye(PK, dtype=jnp.float32),
                   jnp.pad(W2, ((0, 8 - W2.shape[0]), (0, 8 - F2))))
    b1P = jnp.tile(jnp.pad(b1, (0, 8 - b1.shape[0])), PK).reshape(1, 128)
    b2P = jnp.tile(jnp.pad(b2, (0, 8 - F2)), PK).reshape(1, 128)

    degp = _sc_hist(ei3, aux, z1, npad, wn, ag)
    xP3, disP3 = _tc_pack(x_r, degp.reshape(NC, npad // 128, 128), npad)
    xP = xP3.reshape(npadP, 128)
    disP = disP3.reshape(npadP, 128)
    y1P = _tc_pre(xP, W1B, disP, npadP)
    acc1 = _sc_prop(ei3, aux, y1P.reshape(npad, 8), z8, npad, wn, ag, 8)
    y2P = _tc_mid(acc1.reshape(NC, npadP, 128), y1P, disP, W2B, b1P, npadP)
    acc2 = _sc_prop(ei3, aux, y2P.reshape(npad, 8), z8, npad, wn, ag, 8)
    outP = _tc_post(acc2.reshape(NC, npadP, 128), y2P, disP, b2P, npadP)
    out1d = outP.reshape(npad * 8)
    cols = [lax.slice(out1d, (k,), (N * 8,), (8,)) for k in range(F2)]
    return jnp.stack(cols, axis=1)


# speedup vs baseline: 124.9603x; 1.0990x over previous
import functools

import jax
import jax.numpy as jnp
from jax import lax
from jax.experimental import pallas as pl
from jax.experimental.pallas import tpu as pltpu
from jax.experimental.pallas import tpu_sc as plsc

NC = 2
NS = 16
NW = NC * NS
LANE = 128
K = 16
PK = 16


def _sc_hist(ei3, aux, z_tile, npad, wn, ag):
    gpw = wn * K
    rpt = npad // NS

    @functools.partial(
        pl.kernel,
        out_type=jax.ShapeDtypeStruct((NC * npad,), jnp.float32),
        mesh=plsc.VectorSubcoreMesh(core_axis_name="c", subcore_axis_name="s"),
        scratch_types=[
            pltpu.VMEM_SHARED((npad,), jnp.float32),
            pltpu.VMEM((2 * K, 2, LANE), jnp.int32),
            pltpu.VMEM((ag, 2, LANE), jnp.int32),
            pltpu.VMEM((LANE,), jnp.float32),
            pltpu.VMEM((rpt,), jnp.float32),
            pltpu.SemaphoreType.DMA,
            pltpu.SemaphoreType.DMA,
        ],
        compiler_params=pltpu.CompilerParams(use_tc_tiling_on_sc=False),
    )
    def hist(ei_hbm, aux_hbm, z_hbm, out_hbm,
             acc, eibuf, abuf, ones, zbuf, sem, isem):
        cid = lax.axis_index("c")
        sid = lax.axis_index("s")
        w = cid * NS + sid
        base = w * gpw
        pltpu.sync_copy(z_hbm, zbuf)
        pltpu.sync_copy(zbuf, acc.at[pl.ds(sid * rpt, rpt)])
        for t in range(LANE // 16):
            ones[pl.ds(t * 16, 16)] = jnp.ones((16,), jnp.float32)
        pltpu.sync_copy(ei_hbm.at[pl.ds(base, K)], eibuf.at[pl.ds(0, K)])
        plsc.subcore_barrier()

        def body(win, carry):
            p = win & 1

            @pl.when(win != 0)
            def _():
                pltpu.make_async_copy(ei_hbm.at[pl.ds(base, K)],
                                      eibuf.at[pl.ds(p * K, K)], isem).wait()
                for _j in range(K):
                    pltpu.make_async_copy(z_hbm.at[pl.ds(0, LANE)],
                                          ones, sem).wait()

            @pl.when(win + 1 != wn)
            def _():
                pltpu.async_copy(ei_hbm.at[pl.ds(base + (win + 1) * K, K)],
                                 eibuf.at[pl.ds((1 - p) * K, K)], isem)

            for j in range(K):
                pltpu.async_copy(ones, acc.at[eibuf.at[p * K + j].at[1]],
                                 sem, add=True)
            return carry

        lax.fori_loop(0, wn, body, 0)
        for _j in range(K):
            pltpu.make_async_copy(z_hbm.at[pl.ds(0, LANE)], ones, sem).wait()
        pltpu.sync_copy(aux_hbm.at[pl.ds(w * ag, ag)], abuf)
        for j in range(ag):
            pltpu.async_copy(ones, acc.at[abuf.at[j].at[1]], sem, add=True)
        for _j in range(ag):
            pltpu.make_async_copy(z_hbm.at[pl.ds(0, LANE)], ones, sem).wait()
        plsc.subcore_barrier()
        pltpu.sync_copy(acc.at[pl.ds(sid * rpt, rpt)], zbuf)
        pltpu.sync_copy(zbuf, out_hbm.at[pl.ds(cid * npad + sid * rpt, rpt)])

    return hist(ei3, aux, z_tile)


def _sc_prop(ei3, aux, y, z_tile, npad, wn, ag, F):
    gpw = wn * K
    rpt = npad // NS

    @functools.partial(
        pl.kernel,
        out_type=jax.ShapeDtypeStruct((NC * npad, F), jnp.float32),
        mesh=plsc.VectorSubcoreMesh(core_axis_name="c", subcore_axis_name="s"),
        scratch_types=[
            pltpu.VMEM_SHARED((npad, F), jnp.float32),
            pltpu.VMEM((2 * K, 2, LANE), jnp.int32),
            pltpu.VMEM((ag, 2, LANE), jnp.int32),
            pltpu.VMEM((2, K * LANE, F), jnp.float32),
            pltpu.VMEM((rpt // 2, F), jnp.float32),
            pltpu.SemaphoreType.DMA,
            pltpu.SemaphoreType.DMA,
            pltpu.SemaphoreType.DMA,
        ],
        compiler_params=pltpu.CompilerParams(use_tc_tiling_on_sc=False),
    )
    def prop(ei_hbm, aux_hbm, y_hbm, z_hbm, out_hbm,
             acc, eibuf, abuf, gbuf, zbuf, gsem, ssem, isem):
        cid = lax.axis_index("c")
        sid = lax.axis_index("s")
        w = cid * NS + sid
        base = w * gpw
        pltpu.sync_copy(z_hbm, zbuf)
        for c in range(2):
            pltpu.sync_copy(zbuf,
                            acc.at[pl.ds(sid * rpt + c * (rpt // 2), rpt // 2)])
        pltpu.sync_copy(ei_hbm.at[pl.ds(base, K)], eibuf.at[pl.ds(0, K)])
        plsc.subcore_barrier()

        def body(win, carry):
            p = win & 1

            @pl.when(win != 0)
            def _():
                pltpu.make_async_copy(ei_hbm.at[pl.ds(base, K)],
                                      eibuf.at[pl.ds(p * K, K)], isem).wait()

            gd = [
                pltpu.async_copy(y_hbm.at[eibuf.at[p * K + j].at[0]],
                                 gbuf.at[p].at[pl.ds(j * LANE, LANE)], gsem)
                for j in range(K)
            ]

            @pl.when(win != 0)
            def _():
                pltpu.make_async_copy(y_hbm.at[pl.ds(0, K * LANE)],
                                      gbuf.at[1 - p], ssem).wait()

            @pl.when(win + 1 != wn)
            def _():
                pltpu.async_copy(ei_hbm.at[pl.ds(base + (win + 1) * K, K)],
                                 eibuf.at[pl.ds((1 - p) * K, K)], isem)

            for j in range(K):
                gd[j].wait()
                pltpu.async_copy(gbuf.at[p].at[pl.ds(j * LANE, LANE)],
                                 acc.at[eibuf.at[p * K + j].at[1]], ssem,
                                 add=True)
            return carry

        lax.fori_loop(0, wn, body, 0)
        pltpu.make_async_copy(y_hbm.at[pl.ds(0, K * LANE)],
                              gbuf.at[(wn - 1) & 1], ssem).wait()
        pltpu.sync_copy(aux_hbm.at[pl.ds(w * ag, ag)], abuf)
        ad = [
            pltpu.async_copy(y_hbm.at[abuf.at[j].at[0]],
                             gbuf.at[0].at[pl.ds(j * LANE, LANE)], gsem)
            for j in range(ag)
        ]
        for j in range(ag):
            ad[j].wait()
            pltpu.async_copy(gbuf.at[0].at[pl.ds(j * LANE, LANE)],
                             acc.at[abuf.at[j].at[1]], ssem, add=True)
        pltpu.make_async_copy(y_hbm.at[pl.ds(0, ag * LANE)],
                              gbuf.at[0].at[pl.ds(0, ag * LANE)], ssem).wait()
        plsc.subcore_barrier()
        for c in range(2):
            o0 = sid * rpt + c * (rpt // 2)
            pltpu.sync_copy(acc.at[pl.ds(o0, rpt // 2)], zbuf)
            pltpu.sync_copy(zbuf, out_hbm.at[pl.ds(cid * npad + o0, rpt // 2)])

    return prop(ei3, aux, y, z_tile)


_TCB = 800
_BQ = 200


def _tc_pack(x_r, degp3, npad):
    nq = npad // 128

    def body(x_ref, dp_ref, xp_ref, dis_ref):
        deg = dp_ref[0] + dp_ref[1] + 1.0
        dis = lax.rsqrt(deg)
        lane = lax.broadcasted_iota(jnp.int32, (_BQ, 128), 1)
        for s in range(8):
            dis_ref[:, s, :] = jnp.take_along_axis(dis, 16 * s + lane // 8,
                                                   axis=1)
        xq = x_ref[...]
        lane2 = lax.broadcasted_iota(jnp.int32, (4 * _BQ, 128), 1)
        j = lane2 // 8
        k = lane2 % 8
        for h in range(2):
            idx = jnp.minimum(64 * h + 4 * j + k, 127)
            v = jnp.take_along_axis(xq, idx, axis=1)
            xp_ref[:, h, :] = jnp.where(k < 4, v, 0.0)

    return pl.pallas_call(
        body,
        grid=(nq // _BQ,),
        in_specs=[
            pl.BlockSpec((4 * _BQ, 128), lambda i: (i, 0)),
            pl.BlockSpec((NC, _BQ, 128), lambda i: (0, i, 0)),
        ],
        out_specs=[
            pl.BlockSpec((4 * _BQ, 2, 128), lambda i: (i, 0, 0)),
            pl.BlockSpec((_BQ, 8, 128), lambda i: (i, 0, 0)),
        ],
        out_shape=[
            jax.ShapeDtypeStruct((npad // 32, 2, 128), jnp.float32),
            jax.ShapeDtypeStruct((nq, 8, 128), jnp.float32),
        ],
    )(x_r, degp3)


def _tc_pre(xP, W1B, disP, npadP):
    def body(x_ref, w_ref, dis_ref, y1_ref):
        xw = jnp.dot(x_ref[...], w_ref[...], preferred_element_type=jnp.float32)
        y1_ref[...] = xw * dis_ref[...]

    return pl.pallas_call(
        body,
        grid=(npadP // _TCB,),
        in_specs=[
            pl.BlockSpec((_TCB, 128), lambda i: (i, 0)),
            pl.BlockSpec((128, 128), lambda i: (0, 0)),
            pl.BlockSpec((_TCB, 128), lambda i: (i, 0)),
        ],
        out_specs=pl.BlockSpec((_TCB, 128), lambda i: (i, 0)),
        out_shape=jax.ShapeDtypeStruct((npadP, 128), jnp.float32),
    )(xP, W1B, disP)


def _tc_mid(acc1P, y1P, disP, W2B, b1P, npadP):
    def body(a_ref, y1_ref, dis_ref, w_ref, b_ref, y2_ref):
        dis = dis_ref[...]
        h = jnp.maximum(dis * (a_ref[0] + a_ref[1] + y1_ref[...]) + b_ref[...],
                        0.0)
        y2_ref[...] = jnp.dot(h, w_ref[...],
                              preferred_element_type=jnp.float32) * dis

    return pl.pallas_call(
        body,
        grid=(npadP // _TCB,),
        in_specs=[
            pl.BlockSpec((NC, _TCB, 128), lambda i: (0, i, 0)),
            pl.BlockSpec((_TCB, 128), lambda i: (i, 0)),
            pl.BlockSpec((_TCB, 128), lambda i: (i, 0)),
            pl.BlockSpec((128, 128), lambda i: (0, 0)),
            pl.BlockSpec((1, 128), lambda i: (0, 0)),
        ],
        out_specs=pl.BlockSpec((_TCB, 128), lambda i: (i, 0)),
        out_shape=jax.ShapeDtypeStruct((npadP, 128), jnp.float32),
    )(acc1P, y1P, disP, W2B, b1P)


def _tc_post(acc2P, y2P, disP, b2P, npadP):
    def body(a_ref, y2_ref, dis_ref, b_ref, o_ref):
        o_ref[...] = (dis_ref[...] * (a_ref[0] + a_ref[1] + y2_ref[...])
                      + b_ref[...])

    return pl.pallas_call(
        body,
        grid=(npadP // _TCB,),
        in_specs=[
            pl.BlockSpec((NC, _TCB, 128), lambda i: (0, i, 0)),
            pl.BlockSpec((_TCB, 128), lambda i: (i, 0)),
            pl.BlockSpec((_TCB, 128), lambda i: (i, 0)),
            pl.BlockSpec((1, 128), lambda i: (0, 0)),
        ],
        out_specs=pl.BlockSpec((_TCB, 128), lambda i: (i, 0)),
        out_shape=jax.ShapeDtypeStruct((npadP, 128), jnp.float32),
    )(acc2P, y2P, disP, b2P)


def kernel(x, edge_index, W1, b1, W2, b2):
    N = x.shape[0]
    E = edge_index.shape[1]
    npad = 102400 if N == 100000 else ((N + 64 + 16 * _TCB - 1)
                                       // (16 * _TCB)) * (16 * _TCB)
    npadP = npad // PK

    ei = edge_index.astype(jnp.int32)
    G3 = E // LANE
    tail_e = E - G3 * LANE
    ei3 = jnp.transpose(ei[:, :G3 * LANE].reshape(2, G3, LANE), (1, 0, 2))
    gpw = (G3 // NW) // K * K
    wn = gpw // K
    rem = G3 - NW * gpw
    ag = -(-(rem + (1 if tail_e else 0)) // NW)
    auxg = NW * ag
    ar = jnp.arange((auxg - rem) * LANE, dtype=jnp.int32)
    prow = (ar % 64).reshape(auxg - rem, 1, LANE)
    pcol = (N + (ar % 64)).reshape(auxg - rem, 1, LANE)
    padgrp = jnp.concatenate([prow, pcol], axis=1)
    if tail_e:
        tr = jnp.concatenate([ei[0, G3 * LANE:], (ar % 64)[:LANE - tail_e]])
        tc = jnp.concatenate([ei[1, G3 * LANE:],
                              N + (ar % 64)[:LANE - tail_e]])
        padgrp = jnp.concatenate(
            [jnp.stack([tr, tc])[None], padgrp[1:]], axis=0)
    aux = jnp.concatenate([ei3[NW * gpw:], padgrp], axis=0)

    rpt = npad // NS
    z1 = jnp.zeros((rpt,), jnp.float32)
    z8 = jnp.zeros((rpt // 2, 8), jnp.float32)

    x_r = jnp.pad(x.reshape(N * x.shape[1] // 128, 128),
                  ((0, (npad - N) * x.shape[1] // 128), (0, 0)))
    W1B = jnp.kron(jnp.eye(PK, dtype=jnp.float32),
                   jnp.pad(W1, ((0, 8 - W1.shape[0]), (0, 8 - W1.shape[1]))))
    F2 = W2.shape[1]
    W2B = jnp.kron(jnp.eye(PK, dtype=jnp.float32),
                   jnp.pad(W2, ((0, 8 - W2.shape[0]), (0, 8 - F2))))
    b1P = jnp.tile(jnp.pad(b1, (0, 8 - b1.shape[0])), PK).reshape(1, 128)
    b2P = jnp.tile(jnp.pad(b2, (0, 8 - F2)), PK).reshape(1, 128)

    degp = _sc_hist(ei3, aux, z1, npad, wn, ag)
    xP3, disP3 = _tc_pack(x_r, degp.reshape(NC, npad // 128, 128), npad)
    xP = xP3.reshape(npadP, 128)
    disP = disP3.reshape(npadP, 128)
    y1P = _tc_pre(xP, W1B, disP, npadP)
    acc1 = _sc_prop(ei3, aux, y1P.reshape(npad, 8), z8, npad, wn, ag, 8)
    y2P = _tc_mid(acc1.reshape(NC, npadP, 128), y1P, disP, W2B, b1P, npadP)
    acc2 = _sc_prop(ei3, aux, y2P.reshape(npad, 8), z8, npad, wn, ag, 8)
    outP = _tc_post(acc2.reshape(NC, npadP, 128), y2P, disP, b2P, npadP)
    out1d = outP.reshape(npad * 8)
    cols = [lax.slice(out1d, (k,), (N * 8,), (8,)) for k in range(F2)]
    return jnp.stack(cols, axis=1)


# speedup vs baseline: 141.7296x; 1.1342x over previous
import functools

import jax
import jax.numpy as jnp
from jax import lax
from jax.experimental import pallas as pl
from jax.experimental.pallas import tpu as pltpu
from jax.experimental.pallas import tpu_sc as plsc

NC = 2
NS = 16
NW = NC * NS
LANE = 128
K = 16
PK = 16


def _sc_hist(ei3, aux, z_tile, npad, wn, ag):
    gpw = wn * K
    rpt = npad // NS

    @functools.partial(
        pl.kernel,
        out_type=jax.ShapeDtypeStruct((NC * npad,), jnp.float32),
        mesh=plsc.VectorSubcoreMesh(core_axis_name="c", subcore_axis_name="s"),
        scratch_types=[
            pltpu.VMEM_SHARED((npad,), jnp.float32),
            pltpu.VMEM((2 * K, 2, LANE), jnp.int32),
            pltpu.VMEM((ag, 2, LANE), jnp.int32),
            pltpu.VMEM((LANE,), jnp.float32),
            pltpu.VMEM((rpt,), jnp.float32),
            pltpu.SemaphoreType.DMA,
            pltpu.SemaphoreType.DMA,
        ],
        compiler_params=pltpu.CompilerParams(use_tc_tiling_on_sc=False),
    )
    def hist(ei_hbm, aux_hbm, z_hbm, out_hbm,
             acc, eibuf, abuf, ones, zbuf, sem, isem):
        cid = lax.axis_index("c")
        sid = lax.axis_index("s")
        w = cid * NS + sid
        base = w * gpw
        pltpu.sync_copy(z_hbm, zbuf)
        pltpu.sync_copy(zbuf, acc.at[pl.ds(sid * rpt, rpt)])
        for t in range(LANE // 16):
            ones[pl.ds(t * 16, 16)] = jnp.ones((16,), jnp.float32)
        pltpu.sync_copy(ei_hbm.at[pl.ds(base, K)], eibuf.at[pl.ds(0, K)])
        plsc.subcore_barrier()

        def body(win, carry):
            p = win & 1

            @pl.when(win != 0)
            def _():
                pltpu.make_async_copy(ei_hbm.at[pl.ds(base, K)],
                                      eibuf.at[pl.ds(p * K, K)], isem).wait()
                for _j in range(K):
                    pltpu.make_async_copy(z_hbm.at[pl.ds(0, LANE)],
                                          ones, sem).wait()

            @pl.when(win + 1 != wn)
            def _():
                pltpu.async_copy(ei_hbm.at[pl.ds(base + (win + 1) * K, K)],
                                 eibuf.at[pl.ds((1 - p) * K, K)], isem)

            for j in range(K):
                pltpu.async_copy(ones, acc.at[eibuf.at[p * K + j].at[1]],
                                 sem, add=True)
            return carry

        lax.fori_loop(0, wn, body, 0)
        for _j in range(K):
            pltpu.make_async_copy(z_hbm.at[pl.ds(0, LANE)], ones, sem).wait()
        pltpu.sync_copy(aux_hbm.at[pl.ds(w * ag, ag)], abuf)
        for j in range(ag):
            pltpu.async_copy(ones, acc.at[abuf.at[j].at[1]], sem, add=True)
        for _j in range(ag):
            pltpu.make_async_copy(z_hbm.at[pl.ds(0, LANE)], ones, sem).wait()
        plsc.subcore_barrier()
        pltpu.sync_copy(acc.at[pl.ds(sid * rpt, rpt)], zbuf)
        pltpu.sync_copy(zbuf, out_hbm.at[pl.ds(cid * npad + sid * rpt, rpt)])

    return hist(ei3, aux, z_tile)


def _sc_prop(ei3, aux, y, z_tile, npad, wn, ag, F):
    gpw = wn * K
    rpt = npad // NS

    @functools.partial(
        pl.kernel,
        out_type=jax.ShapeDtypeStruct((NC * npad, F), jnp.float32),
        mesh=plsc.VectorSubcoreMesh(core_axis_name="c", subcore_axis_name="s"),
        scratch_types=[
            pltpu.VMEM_SHARED((npad, F), jnp.float32),
            pltpu.VMEM((2 * K, 2, LANE), jnp.int32),
            pltpu.VMEM((ag, 2, LANE), jnp.int32),
            pltpu.VMEM((2, K * LANE, F), jnp.float32),
            pltpu.VMEM((rpt // 2, F), jnp.float32),
            pltpu.SemaphoreType.DMA,
            pltpu.SemaphoreType.DMA,
            pltpu.SemaphoreType.DMA,
        ],
        compiler_params=pltpu.CompilerParams(use_tc_tiling_on_sc=False),
    )
    def prop(ei_hbm, aux_hbm, y_hbm, z_hbm, out_hbm,
             acc, eibuf, abuf, gbuf, zbuf, gsem, ssem, isem):
        cid = lax.axis_index("c")
        sid = lax.axis_index("s")
        w = cid * NS + sid
        base = w * gpw
        pltpu.sync_copy(z_hbm, zbuf)
        for c in range(2):
            pltpu.sync_copy(zbuf,
                            acc.at[pl.ds(sid * rpt + c * (rpt // 2), rpt // 2)])
        pltpu.sync_copy(ei_hbm.at[pl.ds(base, K)], eibuf.at[pl.ds(0, K)])
        plsc.subcore_barrier()

        def body(win, carry):
            p = win & 1

            @pl.when(win != 0)
            def _():
                pltpu.make_async_copy(ei_hbm.at[pl.ds(base, K)],
                                      eibuf.at[pl.ds(p * K, K)], isem).wait()

            gd = [
                pltpu.async_copy(y_hbm.at[eibuf.at[p * K + j].at[0]],
                                 gbuf.at[p].at[pl.ds(j * LANE, LANE)], gsem)
                for j in range(K)
            ]

            @pl.when(win != 0)
            def _():
                pltpu.make_async_copy(y_hbm.at[pl.ds(0, K * LANE)],
                                      gbuf.at[1 - p], ssem).wait()

            @pl.when(win + 1 != wn)
            def _():
                pltpu.async_copy(ei_hbm.at[pl.ds(base + (win + 1) * K, K)],
                                 eibuf.at[pl.ds((1 - p) * K, K)], isem)

            for j in range(K):
                gd[j].wait()
                pltpu.async_copy(gbuf.at[p].at[pl.ds(j * LANE, LANE)],
                                 acc.at[eibuf.at[p * K + j].at[1]], ssem,
                                 add=True)
            return carry

        lax.fori_loop(0, wn, body, 0)
        pltpu.make_async_copy(y_hbm.at[pl.ds(0, K * LANE)],
                              gbuf.at[(wn - 1) & 1], ssem).wait()
        pltpu.sync_copy(aux_hbm.at[pl.ds(w * ag, ag)], abuf)
        ad = [
            pltpu.async_copy(y_hbm.at[abuf.at[j].at[0]],
                             gbuf.at[0].at[pl.ds(j * LANE, LANE)], gsem)
            for j in range(ag)
        ]
        for j in range(ag):
            ad[j].wait()
            pltpu.async_copy(gbuf.at[0].at[pl.ds(j * LANE, LANE)],
                             acc.at[abuf.at[j].at[1]], ssem, add=True)
        pltpu.make_async_copy(y_hbm.at[pl.ds(0, ag * LANE)],
                              gbuf.at[0].at[pl.ds(0, ag * LANE)], ssem).wait()
        plsc.subcore_barrier()
        for c in range(2):
            o0 = sid * rpt + c * (rpt // 2)
            pltpu.sync_copy(acc.at[pl.ds(o0, rpt // 2)], zbuf)
            pltpu.sync_copy(zbuf, out_hbm.at[pl.ds(cid * npad + o0, rpt // 2)])

    return prop(ei3, aux, y, z_tile)


_TCB = 800
_BQ = 200


def _tc_pack(x_r, degp3, npad):
    nq = npad // 128

    def body(x_ref, dp_ref, xp_ref, dis_ref):
        deg = dp_ref[0] + dp_ref[1] + 1.0
        dis = lax.rsqrt(deg)
        lane = lax.broadcasted_iota(jnp.int32, (_BQ, 128), 1)
        xtb = x_ref[...]
        lmod = lane % 8
        for s in range(8):
            idx = 16 * s + lane // 8
            dis_ref[:, s, :] = jnp.take_along_axis(dis, idx, axis=1)
            acc = jnp.zeros((_BQ, 128), jnp.float32)
            for k in range(4):
                vk = jnp.take_along_axis(xtb[k], idx, axis=1)
                acc = jnp.where(lmod == k, vk, acc)
            xp_ref[:, s, :] = acc

    return pl.pallas_call(
        body,
        grid=(nq // _BQ,),
        in_specs=[
            pl.BlockSpec((4, _BQ, 128), lambda i: (0, i, 0)),
            pl.BlockSpec((NC, _BQ, 128), lambda i: (0, i, 0)),
        ],
        out_specs=[
            pl.BlockSpec((_BQ, 8, 128), lambda i: (i, 0, 0)),
            pl.BlockSpec((_BQ, 8, 128), lambda i: (i, 0, 0)),
        ],
        out_shape=[
            jax.ShapeDtypeStruct((nq, 8, 128), jnp.float32),
            jax.ShapeDtypeStruct((nq, 8, 128), jnp.float32),
        ],
    )(x_r, degp3)


def _tc_pre(xP, W1B, disP, npadP):
    def body(x_ref, w_ref, dis_ref, y1_ref):
        xw = jnp.dot(x_ref[...], w_ref[...], preferred_element_type=jnp.float32)
        y1_ref[...] = xw * dis_ref[...]

    return pl.pallas_call(
        body,
        grid=(npadP // _TCB,),
        in_specs=[
            pl.BlockSpec((_TCB, 128), lambda i: (i, 0)),
            pl.BlockSpec((128, 128), lambda i: (0, 0)),
            pl.BlockSpec((_TCB, 128), lambda i: (i, 0)),
        ],
        out_specs=pl.BlockSpec((_TCB, 128), lambda i: (i, 0)),
        out_shape=jax.ShapeDtypeStruct((npadP, 128), jnp.float32),
    )(xP, W1B, disP)


def _tc_mid(acc1P, y1P, disP, W2B, b1P, npadP):
    def body(a_ref, y1_ref, dis_ref, w_ref, b_ref, y2_ref):
        dis = dis_ref[...]
        h = jnp.maximum(dis * (a_ref[0] + a_ref[1] + y1_ref[...]) + b_ref[...],
                        0.0)
        y2_ref[...] = jnp.dot(h, w_ref[...],
                              preferred_element_type=jnp.float32) * dis

    return pl.pallas_call(
        body,
        grid=(npadP // _TCB,),
        in_specs=[
            pl.BlockSpec((NC, _TCB, 128), lambda i: (0, i, 0)),
            pl.BlockSpec((_TCB, 128), lambda i: (i, 0)),
            pl.BlockSpec((_TCB, 128), lambda i: (i, 0)),
            pl.BlockSpec((128, 128), lambda i: (0, 0)),
            pl.BlockSpec((1, 128), lambda i: (0, 0)),
        ],
        out_specs=pl.BlockSpec((_TCB, 128), lambda i: (i, 0)),
        out_shape=jax.ShapeDtypeStruct((npadP, 128), jnp.float32),
    )(acc1P, y1P, disP, W2B, b1P)


def _tc_post(acc2P, y2P, disP, b2P, npadP):
    def body(a_ref, y2_ref, dis_ref, b_ref, o_ref):
        o_ref[...] = (dis_ref[...] * (a_ref[0] + a_ref[1] + y2_ref[...])
                      + b_ref[...])

    return pl.pallas_call(
        body,
        grid=(npadP // _TCB,),
        in_specs=[
            pl.BlockSpec((NC, _TCB, 128), lambda i: (0, i, 0)),
            pl.BlockSpec((_TCB, 128), lambda i: (i, 0)),
            pl.BlockSpec((_TCB, 128), lambda i: (i, 0)),
            pl.BlockSpec((1, 128), lambda i: (0, 0)),
        ],
        out_specs=pl.BlockSpec((_TCB, 128), lambda i: (i, 0)),
        out_shape=jax.ShapeDtypeStruct((npadP, 128), jnp.float32),
    )(acc2P, y2P, disP, b2P)


def kernel(x, edge_index, W1, b1, W2, b2):
    N = x.shape[0]
    E = edge_index.shape[1]
    npad = 102400 if N == 100000 else ((N + 64 + 16 * _TCB - 1)
                                       // (16 * _TCB)) * (16 * _TCB)
    npadP = npad // PK

    ei = edge_index.astype(jnp.int32)
    G3 = E // LANE
    tail_e = E - G3 * LANE
    ei3 = jnp.transpose(ei[:, :G3 * LANE].reshape(2, G3, LANE), (1, 0, 2))
    gpw = (G3 // NW) // K * K
    wn = gpw // K
    rem = G3 - NW * gpw
    ag = -(-(rem + (1 if tail_e else 0)) // NW)
    auxg = NW * ag
    ar = jnp.arange((auxg - rem) * LANE, dtype=jnp.int32)
    prow = (ar % 64).reshape(auxg - rem, 1, LANE)
    pcol = (N + (ar % 64)).reshape(auxg - rem, 1, LANE)
    padgrp = jnp.concatenate([prow, pcol], axis=1)
    if tail_e:
        tr = jnp.concatenate([ei[0, G3 * LANE:], (ar % 64)[:LANE - tail_e]])
        tc = jnp.concatenate([ei[1, G3 * LANE:],
                              N + (ar % 64)[:LANE - tail_e]])
        padgrp = jnp.concatenate(
            [jnp.stack([tr, tc])[None], padgrp[1:]], axis=0)
    aux = jnp.concatenate([ei3[NW * gpw:], padgrp], axis=0)

    rpt = npad // NS
    z1 = jnp.zeros((rpt,), jnp.float32)
    z8 = jnp.zeros((rpt // 2, 8), jnp.float32)

    x_r = jnp.pad(x.T, ((0, 0), (0, npad - N))).reshape(
        x.shape[1], npad // 128, 128)
    W1B = jnp.kron(jnp.eye(PK, dtype=jnp.float32),
                   jnp.pad(W1, ((0, 8 - W1.shape[0]), (0, 8 - W1.shape[1]))))
    F2 = W2.shape[1]
    W2B = jnp.kron(jnp.eye(PK, dtype=jnp.float32),
                   jnp.pad(W2, ((0, 8 - W2.shape[0]), (0, 8 - F2))))
    b1P = jnp.tile(jnp.pad(b1, (0, 8 - b1.shape[0])), PK).reshape(1, 128)
    b2P = jnp.tile(jnp.pad(b2, (0, 8 - F2)), PK).reshape(1, 128)

    degp = _sc_hist(ei3, aux, z1, npad, wn, ag)
    xP3, disP3 = _tc_pack(x_r, degp.reshape(NC, npad // 128, 128), npad)
    xP = xP3.reshape(npadP, 128)
    disP = disP3.reshape(npadP, 128)
    y1P = _tc_pre(xP, W1B, disP, npadP)
    acc1 = _sc_prop(ei3, aux, y1P.reshape(npad, 8), z8, npad, wn, ag, 8)
    y2P = _tc_mid(acc1.reshape(NC, npadP, 128), y1P, disP, W2B, b1P, npadP)
    acc2 = _sc_prop(ei3, aux, y2P.reshape(npad, 8), z8, npad, wn, ag, 8)
    outP = _tc_post(acc2.reshape(NC, npadP, 128), y2P, disP, b2P, npadP)
    out1d = outP.reshape(npad * 8)
    cols = [lax.slice(out1d, (k,), (N * 8,), (8,)) for k in range(F2)]
    return jnp.stack(cols, axis=1)
